# baseline (speedup 1.0000x reference)
import functools
import jax
import jax.numpy as jnp
from jax import lax
from jax.experimental import pallas as pl
from jax.experimental.pallas import tpu as pltpu
from jax.experimental.pallas import tpu_sc as plsc

D = 64
NC = 2
NS = 16
CH = 128


def _ceil_to(x, m):
    return (x + m - 1) // m * m


@functools.lru_cache(maxsize=None)
def _make_segsum(nnz_pad, h, edge_part):
    nworkers = NC * NS if edge_part else NS
    epp = nnz_pad // nworkers
    cps = epp // CH
    rps = h // NS
    zslices = (h + 128) // 128
    mesh = plsc.VectorSubcoreMesh(core_axis_name="c", subcore_axis_name="s")
    out_sds = (jax.ShapeDtypeStruct((NC, h, D), jnp.float32) if edge_part
               else jax.ShapeDtypeStruct((2 * h, D), jnp.float32))

    @functools.partial(
        pl.kernel,
        out_type=out_sds,
        mesh=mesh,
        scratch_types=[
            pltpu.VMEM((CH,), jnp.int32), pltpu.VMEM((CH,), jnp.int32),
            pltpu.VMEM((CH,), jnp.float32), pltpu.VMEM((CH,), jnp.float32),
            pltpu.VMEM((CH,), jnp.int32), pltpu.VMEM((CH,), jnp.int32),
            pltpu.VMEM((CH, D), jnp.float32), pltpu.VMEM((CH, D), jnp.float32),
            pltpu.VMEM_SHARED((h + 128, D), jnp.float32),
            pltpu.SemaphoreType.DMA, pltpu.SemaphoreType.DMA,
            pltpu.SemaphoreType.DMA, pltpu.SemaphoreType.DMA,
            pltpu.SemaphoreType.DMA, pltpu.SemaphoreType.DMA,
        ],
        compiler_params=pltpu.CompilerParams(use_tc_tiling_on_sc=False,
                                             needs_layout_passes=False),
    )
    def seg_kernel(scat_hbm, gath_hbm, vals_hbm, table_hbm, out_hbm,
                   gi0, gi1, vv0, vv1, si0, si1, ro0, ro1, shared,
                   semi0, semi1, semg0, semg1, sems0, sems1):
        cid = lax.axis_index("c")
        sid = lax.axis_index("s")
        base = 0 if edge_part else cid * h
        col_ids = [lax.iota(jnp.int32, 16) + 16 * d4 for d4 in range(4)]
        gis, vvs, sis, ros = (gi0, gi1), (vv0, vv1), (si0, si1), (ro0, ro1)
        semis, semgs, semss = (semi0, semi1), (semg0, semg1), (sems0, sems1)

        def zrow(r, _):
            ev = lax.broadcast(r, (16,))
            for d4 in range(4):
                plsc.store_scatter(ro0, [ev, col_ids[d4]],
                                   jnp.zeros((16,), jnp.float32))
            return _
        lax.fori_loop(0, CH, zrow, None)

        def zshared(i, _):
            s = sid + i * NS

            @pl.when(s < zslices)
            def _():
                pltpu.sync_copy(ro0, shared.at[pl.ds(s * 128, 128)])
            return _
        lax.fori_loop(0, (zslices + NS - 1) // NS, zshared, None)
        plsc.subcore_barrier()

        eoff = ((cid * NS + sid) if edge_part else sid) * epp

        def issue_idx(j, b):
            off = eoff + j * CH
            pltpu.async_copy(gath_hbm.at[pl.ds(off, CH)], gis[b], semis[b])
            pltpu.async_copy(vals_hbm.at[pl.ds(off, CH)], vvs[b], semis[b])
            pltpu.async_copy(scat_hbm.at[pl.ds(off, CH)], sis[b], semis[b])

        def drain_idx(b):
            pltpu.make_async_copy(gath_hbm.at[pl.ds(0, CH)], gis[b], semis[b]).wait()
            pltpu.make_async_copy(vals_hbm.at[pl.ds(0, CH)], vvs[b], semis[b]).wait()
            pltpu.make_async_copy(scat_hbm.at[pl.ds(0, CH)], sis[b], semis[b]).wait()

        def drain_scat(b):
            pltpu.make_async_copy(ros[b], shared.at[sis[b]], semss[b]).wait()

        def process(b):
            pltpu.make_async_copy(table_hbm.at[gis[b]], ros[b], semgs[b]).wait()

            def scale(e, _):
                ev = lax.broadcast(e, (16,))
                vs = plsc.load_gather(vvs[b], [ev])
                for d4 in range(4):
                    x = plsc.load_gather(ros[b], [ev, col_ids[d4]])
                    plsc.store_scatter(ros[b], [ev, col_ids[d4]], x * vs)
                return _
            lax.fori_loop(0, CH, scale, None)
            for g in range(CH // 16):
                loc = sis[b][pl.ds(g * 16, 16)] - base
                ok = (loc >= 0) & (loc < h)
                sis[b][pl.ds(g * 16, 16)] = jnp.where(ok, loc, h)
            pltpu.async_copy(ros[b], shared.at[sis[b]], semss[b], add=True)

        issue_idx(0, 0)
        issue_idx(1, 1)

        def pair(p, _):
            drain_idx(0)

            @pl.when(p > 0)
            def _():
                drain_scat(0)
            pltpu.async_copy(table_hbm.at[gis[0]], ros[0], semgs[0])
            drain_idx(1)

            @pl.when(p > 0)
            def _():
                drain_scat(1)
            pltpu.async_copy(table_hbm.at[gis[1]], ros[1], semgs[1])
            process(0)

            @pl.when(2 * p + 2 < cps)
            def _():
                issue_idx(2 * p + 2, 0)
            process(1)

            @pl.when(2 * p + 3 < cps)
            def _():
                issue_idx(2 * p + 3, 1)
            return _
        lax.fori_loop(0, cps // 2, pair, None)
        drain_scat(0)
        drain_scat(1)
        plsc.subcore_barrier()
        if edge_part:
            pltpu.sync_copy(shared.at[pl.ds(sid * rps, rps)],
                            out_hbm.at[cid, pl.ds(sid * rps, rps)])
        else:
            pltpu.sync_copy(shared.at[pl.ds(sid * rps, rps)],
                            out_hbm.at[pl.ds(cid * h + sid * rps, rps)])

    return seg_kernel


@functools.lru_cache(maxsize=None)
def _make_front(p_rui, p_rgu, p_rgi, h_u, h_i, h_g, n_mem_pad):
    mesh = plsc.VectorSubcoreMesh(core_axis_name="c", subcore_axis_name="s")
    h_max = max(h_u, h_i, h_g)
    bpw = n_mem_pad // (NC * NS)

    out_types = (
        jax.ShapeDtypeStruct((2 * h_u, D), jnp.float32),
        jax.ShapeDtypeStruct((2 * h_u, D), jnp.float32),
        jax.ShapeDtypeStruct((NC, h_i, D), jnp.float32),
        jax.ShapeDtypeStruct((NC, h_i, D), jnp.float32),
        jax.ShapeDtypeStruct((NC, h_g, D), jnp.float32),
        jax.ShapeDtypeStruct((NC, h_g, D), jnp.float32),
        jax.ShapeDtypeStruct((n_mem_pad, D), jnp.float32),
    )

    @functools.partial(
        pl.kernel,
        out_type=out_types,
        mesh=mesh,
        scratch_types=[
            pltpu.VMEM((CH,), jnp.int32), pltpu.VMEM((CH,), jnp.int32),
            pltpu.VMEM((CH,), jnp.float32), pltpu.VMEM((CH,), jnp.float32),
            pltpu.VMEM((CH,), jnp.int32), pltpu.VMEM((CH,), jnp.int32),
            pltpu.VMEM((CH, D), jnp.float32), pltpu.VMEM((CH, D), jnp.float32),
            pltpu.VMEM((CH, D), jnp.float32),
            pltpu.VMEM((bpw,), jnp.int32),
            pltpu.VMEM_SHARED((h_max + 128, D), jnp.float32),
            pltpu.SemaphoreType.DMA, pltpu.SemaphoreType.DMA,
            pltpu.SemaphoreType.DMA, pltpu.SemaphoreType.DMA,
            pltpu.SemaphoreType.DMA, pltpu.SemaphoreType.DMA,
        ],
        compiler_params=pltpu.CompilerParams(use_tc_tiling_on_sc=False,
                                             needs_layout_passes=False),
    )
    def front_kernel(rui_r, rui_c, rui_v, rgu_r, rgu_c, rgu_v,
                     rgi_r, rgi_c, rgi_v, item_emb, user_emb, group_emb,
                     mem_idx_hbm,
                     o_rui_ei, o_rgu_t_eg, o_rui_t_eu, o_rgi_t_eg,
                     o_rgi_ei, o_rgu_eu, o_me,
                     gi0, gi1, vv0, vv1, si0, si1, ro0, ro1, zbuf,
                     midx, shared,
                     semi0, semi1, semg0, semg1, sems0, sems1):
        cid = lax.axis_index("c")
        sid = lax.axis_index("s")
        col_ids = [lax.iota(jnp.int32, 16) + 16 * d4 for d4 in range(4)]
        gis, vvs, sis, ros = (gi0, gi1), (vv0, vv1), (si0, si1), (ro0, ro1)
        semis, semgs, semss = (semi0, semi1), (semg0, semg1), (sems0, sems1)

        def zrow(r, _):
            ev = lax.broadcast(r, (16,))
            for d4 in range(4):
                plsc.store_scatter(zbuf, [ev, col_ids[d4]],
                                   jnp.zeros((16,), jnp.float32))
            return _
        lax.fori_loop(0, CH, zrow, None)

        mbase = (cid * NS + sid) * bpw
        pltpu.sync_copy(mem_idx_hbm.at[pl.ds(mbase, bpw)], midx)
        pltpu.async_copy(user_emb.at[midx], ro0.at[pl.ds(0, bpw)], semg0).wait()
        pltpu.sync_copy(ro0.at[pl.ds(0, bpw)], o_me.at[pl.ds(mbase, bpw)])

        def seg_phase(scat_hbm, gath_hbm, vals_hbm, table_hbm, out_hbm,
                      h, nnz_pad, edge_part):
            epp = nnz_pad // (NC * NS if edge_part else NS)
            cps = epp // CH
            rps = h // NS
            zslices = (h + 128) // 128
            base = 0 if edge_part else cid * h

            def zsh(i, _):
                s = sid + i * NS

                @pl.when(s < zslices)
                def _():
                    pltpu.sync_copy(zbuf, shared.at[pl.ds(s * 128, 128)])
                return _
            lax.fori_loop(0, (zslices + NS - 1) // NS, zsh, None)
            plsc.subcore_barrier()

            eoff = ((cid * NS + sid) if edge_part else sid) * epp

            def issue_idx(j, b):
                off = eoff + j * CH
                pltpu.async_copy(gath_hbm.at[pl.ds(off, CH)], gis[b], semis[b])
                pltpu.async_copy(vals_hbm.at[pl.ds(off, CH)], vvs[b], semis[b])
                pltpu.async_copy(scat_hbm.at[pl.ds(off, CH)], sis[b], semis[b])

            def drain_idx(b):
                pltpu.make_async_copy(gath_hbm.at[pl.ds(0, CH)], gis[b], semis[b]).wait()
                pltpu.make_async_copy(vals_hbm.at[pl.ds(0, CH)], vvs[b], semis[b]).wait()
                pltpu.make_async_copy(scat_hbm.at[pl.ds(0, CH)], sis[b], semis[b]).wait()

            def drain_scat(b):
                pltpu.make_async_copy(ros[b], shared.at[sis[b]], semss[b]).wait()

            def process(b):
                pltpu.make_async_copy(table_hbm.at[gis[b]], ros[b], semgs[b]).wait()

                def scale(q, _):
                    for u in range(4):
                        e = q * 4 + u
                        ev = lax.broadcast(e, (16,))
                        vs = plsc.load_gather(vvs[b], [ev])
                        for d4 in range(4):
                            x = plsc.load_gather(ros[b], [ev, col_ids[d4]])
                            plsc.store_scatter(ros[b], [ev, col_ids[d4]], x * vs)
                    return _
                lax.fori_loop(0, CH // 4, scale, None)
                for g in range(CH // 16):
                    loc = sis[b][pl.ds(g * 16, 16)] - base
                    ok = (loc >= 0) & (loc < h)
                    sis[b][pl.ds(g * 16, 16)] = jnp.where(ok, loc, h)
                pltpu.async_copy(ros[b], shared.at[sis[b]], semss[b], add=True)

            issue_idx(0, 0)
            issue_idx(1, 1)

            def pair(p, _):
                drain_idx(0)

                @pl.when(p > 0)
                def _():
                    drain_scat(0)
                pltpu.async_copy(table_hbm.at[gis[0]], ros[0], semgs[0])
                drain_idx(1)

                @pl.when(p > 0)
                def _():
                    drain_scat(1)
                pltpu.async_copy(table_hbm.at[gis[1]], ros[1], semgs[1])
                process(0)

                @pl.when(2 * p + 2 < cps)
                def _():
                    issue_idx(2 * p + 2, 0)
                process(1)

                @pl.when(2 * p + 3 < cps)
                def _():
                    issue_idx(2 * p + 3, 1)
                return _
            lax.fori_loop(0, cps // 2, pair, None)
            drain_scat(0)
            drain_scat(1)
            plsc.subcore_barrier()
            if edge_part:
                pltpu.sync_copy(shared.at[pl.ds(sid * rps, rps)],
                                out_hbm.at[cid, pl.ds(sid * rps, rps)])
            else:
                pltpu.sync_copy(shared.at[pl.ds(sid * rps, rps)],
                                out_hbm.at[pl.ds(cid * h + sid * rps, rps)])
            plsc.subcore_barrier()

        seg_phase(rui_r, rui_c, rui_v, item_emb, o_rui_ei, h_u, p_rui, False)
        seg_phase(rgu_c, rgu_r, rgu_v, group_emb, o_rgu_t_eg, h_u, p_rgu, False)
        seg_phase(rui_c, rui_r, rui_v, user_emb, o_rui_t_eu, h_i, p_rui, True)
        seg_phase(rgi_c, rgi_r, rgi_v, group_emb, o_rgi_t_eg, h_i, p_rgi, True)
        seg_phase(rgi_r, rgi_c, rgi_v, item_emb, o_rgi_ei, h_g, p_rgi, True)
        seg_phase(rgu_r, rgu_c, rgu_v, user_emb, o_rgu_eu, h_g, p_rgu, True)

    return front_kernel


def _pad_edges(rows, cols, vals, mult):
    nnz = rows.shape[0]
    nnz_pad = _ceil_to(nnz, mult)
    pad = nnz_pad - nnz
    if pad:
        rows = jnp.concatenate([rows, jnp.zeros((pad,), jnp.int32)])
        cols = jnp.concatenate([cols, jnp.zeros((pad,), jnp.int32)])
        vals = jnp.concatenate([vals, jnp.zeros((pad,), jnp.float32)])
    return rows, cols, vals, nnz_pad


def _segsum(scat_idx, gath_idx, vals, table, n_rows):
    edge_part = n_rows <= 12288
    nnz = scat_idx.shape[0]
    nnz_pad = _ceil_to(nnz, 2 * CH * (NC * NS if edge_part else NS))
    pad = nnz_pad - nnz
    if pad:
        scat_idx = jnp.concatenate([scat_idx, jnp.zeros((pad,), jnp.int32)])
        gath_idx = jnp.concatenate([gath_idx, jnp.zeros((pad,), jnp.int32)])
        vals = jnp.concatenate([vals, jnp.zeros((pad,), jnp.float32)])
    if edge_part:
        h = _ceil_to(n_rows, 128)
        out = _make_segsum(nnz_pad, h, True)(scat_idx, gath_idx, vals, table)
        return [out[0, :n_rows], out[1, :n_rows]]
    h = _ceil_to(_ceil_to(n_rows, 2) // 2, 128)
    out = _make_segsum(nnz_pad, h, False)(scat_idx, gath_idx, vals, table)
    return [out[:n_rows]]


IB = 1024


def _att_colsum(item_pad, me_pad, n_items):
    ip, _ = item_pad.shape
    mp, _ = me_pad.shape
    nb = ip // IB

    def body(x_ref, me_ref, s_ref):
        @pl.when(pl.program_id(0) == 0)
        def _():
            s_ref[...] = jnp.zeros_like(s_ref)
        e = lax.dot_general(x_ref[...], me_ref[...], (((1,), (1,)), ((), ())),
                            preferred_element_type=jnp.float32)
        rid = pl.program_id(0) * IB + lax.broadcasted_iota(jnp.int32, (IB, 1), 0)
        contrib = jnp.where(rid < n_items, jnp.exp(e), 0.0)
        s_ref[...] += jnp.sum(contrib, axis=0, keepdims=True)

    return pl.pallas_call(
        body,
        grid=(nb,),
        in_specs=[pl.BlockSpec((IB, D), lambda i: (i, 0)),
                  pl.BlockSpec((mp, D), lambda i: (0, 0))],
        out_specs=pl.BlockSpec((1, mp), lambda i: (0, 0)),
        out_shape=jax.ShapeDtypeStruct((1, mp), jnp.float32),
    )(item_pad, me_pad)


def _att_apply(item_pad, me_pad, colsum):
    ip, _ = item_pad.shape
    mp, _ = me_pad.shape
    nb = ip // IB

    def body(x_ref, me_ref, s_ref, o_ref):
        x = x_ref[...]
        e = lax.dot_general(x, me_ref[...], (((1,), (1,)), ((), ())),
                            preferred_element_type=jnp.float32)
        w = jnp.exp(e) / s_ref[...]
        att = lax.dot_general(w, me_ref[...], (((1,), (0,)), ((), ())),
                              preferred_element_type=jnp.float32)
        o_ref[...] = att * x

    return pl.pallas_call(
        body,
        grid=(nb,),
        in_specs=[pl.BlockSpec((IB, D), lambda i: (i, 0)),
                  pl.BlockSpec((mp, D), lambda i: (0, 0)),
                  pl.BlockSpec((1, mp), lambda i: (0, 0))],
        out_specs=pl.BlockSpec((IB, D), lambda i: (i, 0)),
        out_shape=jax.ShapeDtypeStruct((ip, D), jnp.float32),
    )(item_pad, me_pad, colsum)


RB = 512


def _finish(acc):
    y = jnp.where(acc >= 0, acc, 0.01 * acc)
    nrm = jnp.sqrt(jnp.sum(y * y, axis=1, keepdims=True))
    return y / jnp.maximum(nrm, 1e-12)


def _dlin(x, w_ref, k):
    return lax.dot_general(x, w_ref[k], (((1,), (1,)), ((), ())),
                           preferred_element_type=jnp.float32)


def _combine(base, part_lists, W, bias, n_rows, group_pattern):
    np_ = _ceil_to(n_rows, RB)

    def padr(z):
        return jnp.pad(z, ((0, np_ - n_rows), (0, 0)))

    counts = [len(pl_) for pl_ in part_lists]

    def body(*refs):
        x = refs[0][...]
        pos = 1
        aggs = []
        for c in counts:
            agg = refs[pos][...]
            for r in refs[pos + 1:pos + c]:
                agg = agg + r[...]
            aggs.append(agg)
            pos += c
        w_ref, bias_ref, o_ref = refs[pos], refs[pos + 1], refs[pos + 2]
        acc = jnp.sum(bias_ref[...], axis=0, keepdims=True)
        if group_pattern:
            a, b, c = aggs
            acc = (acc + _dlin(x, w_ref, 0) + _dlin(a, w_ref, 1)
                   + _dlin(b * x, w_ref, 2) + _dlin(a * x, w_ref, 3)
                   + _dlin(c, w_ref, 4))
        else:
            a, b = aggs
            acc = (acc + _dlin(x, w_ref, 0) + _dlin(a, w_ref, 1)
                   + _dlin(a * x, w_ref, 2) + _dlin(b * x, w_ref, 3)
                   + _dlin(b, w_ref, 4))
        o_ref[...] = _finish(acc)

    flat_parts = [p for pl_ in part_lists for p in pl_]
    n_data = 1 + len(flat_parts)
    out = pl.pallas_call(
        body,
        grid=(np_ // RB,),
        in_specs=[pl.BlockSpec((RB, D), lambda i: (i, 0))] * n_data
        + [pl.BlockSpec((5, D, D), lambda i: (0, 0, 0)),
           pl.BlockSpec((5, D), lambda i: (0, 0))],
        out_specs=pl.BlockSpec((RB, D), lambda i: (i, 0)),
        out_shape=jax.ShapeDtypeStruct((np_, D), jnp.float32),
    )(padr(base), *[padr(p) for p in flat_parts], W, bias)
    return out[:n_rows]


def kernel(group_embedding, user_embedding, item_embedding, members,
           rui_rows, rui_cols, rui_vals, rgu_rows, rgu_cols, rgu_vals,
           rgi_rows, rgi_cols, rgi_vals, Wg, bg, Wu, bu, Wi, bi):
    G, U, I = group_embedding.shape[0], user_embedding.shape[0], item_embedding.shape[0]

    h_u = _ceil_to(_ceil_to(U, 2) // 2, 128)
    h_i = _ceil_to(I, 128)
    h_g = _ceil_to(G, 128)
    rui_r, rui_c, rui_v, p_rui = _pad_edges(rui_rows, rui_cols, rui_vals, 2 * CH * NC * NS)
    rgu_r, rgu_c, rgu_v, p_rgu = _pad_edges(rgu_rows, rgu_cols, rgu_vals, 2 * CH * NC * NS)
    rgi_r, rgi_c, rgi_v, p_rgi = _pad_edges(rgi_rows, rgi_cols, rgi_vals, 2 * CH * NC * NS)
    mflat = members.reshape(-1).astype(jnp.int32)
    n_mem = mflat.shape[0]
    mp = _ceil_to(n_mem, 8 * NC * NS)
    if mp != n_mem:
        mflat = jnp.concatenate([mflat, jnp.zeros((mp - n_mem,), jnp.int32)])

    front = _make_front(p_rui, p_rgu, p_rgi, h_u, h_i, h_g, mp)
    (rui_ei_f, rgu_t_eg_f, rui_t_eu_p, rgi_t_eg_p, rgi_ei_p, rgu_eu_p, me) = front(
        rui_r, rui_c, rui_v, rgu_r, rgu_c, rgu_v, rgi_r, rgi_c, rgi_v,
        item_embedding, user_embedding, group_embedding, mflat)
    rui_ei = [rui_ei_f[:U]]
    rgu_t_eg = [rgu_t_eg_f[:U]]
    rui_t_eu = [rui_t_eu_p[0, :I], rui_t_eu_p[1, :I]]
    rgi_t_eg = [rgi_t_eg_p[0, :I], rgi_t_eg_p[1, :I]]
    rgi_ei = [rgi_ei_p[0, :G], rgi_ei_p[1, :G]]
    rgu_eu = [rgu_eu_p[0, :G], rgu_eu_p[1, :G]]
    me_pad = jnp.where(
        (jnp.arange(mp) < n_mem)[:, None], me, 0.0)
    ip = _ceil_to(I, IB)
    item_pad = jnp.pad(item_embedding, ((0, ip - I), (0, 0)))
    colsum = _att_colsum(item_pad, me_pad, I)
    attentive = _att_apply(item_pad, me_pad, colsum)

    atten_g = _segsum(rgi_rows, rgi_cols, rgi_vals, attentive, G)

    nu = _combine(user_embedding, [rui_ei, rgu_t_eg], Wu, bu, U, False)
    ni = _combine(item_embedding, [rui_t_eu, rgi_t_eg], Wi, bi, I, False)
    ng = _combine(group_embedding, [rgi_ei, rgu_eu, atten_g], Wg, bg, G, True)
    return ng, nu, ni


# speedup vs baseline: 1.3798x; 1.3798x over previous
import functools
import jax
import jax.numpy as jnp
from jax import lax
from jax.experimental import pallas as pl
from jax.experimental.pallas import tpu as pltpu
from jax.experimental.pallas import tpu_sc as plsc

D = 64
NC = 2
NS = 16
CH = 128


def _ceil_to(x, m):
    return (x + m - 1) // m * m


@functools.lru_cache(maxsize=None)
def _make_segsum(nnz_pad, h, edge_part):
    nworkers = NC * NS if edge_part else NS
    epp = nnz_pad // nworkers
    cps = epp // CH
    rps = h // NS
    zslices = (h + 128) // 128
    mesh = plsc.VectorSubcoreMesh(core_axis_name="c", subcore_axis_name="s")
    out_sds = (jax.ShapeDtypeStruct((NC, h, D), jnp.float32) if edge_part
               else jax.ShapeDtypeStruct((2 * h, D), jnp.float32))

    @functools.partial(
        pl.kernel,
        out_type=out_sds,
        mesh=mesh,
        scratch_types=[
            pltpu.VMEM((CH,), jnp.int32), pltpu.VMEM((CH,), jnp.int32),
            pltpu.VMEM((CH,), jnp.float32), pltpu.VMEM((CH,), jnp.float32),
            pltpu.VMEM((CH,), jnp.int32), pltpu.VMEM((CH,), jnp.int32),
            pltpu.VMEM((CH, D), jnp.float32), pltpu.VMEM((CH, D), jnp.float32),
            pltpu.VMEM_SHARED((h + 128, D), jnp.float32),
            pltpu.SemaphoreType.DMA, pltpu.SemaphoreType.DMA,
            pltpu.SemaphoreType.DMA, pltpu.SemaphoreType.DMA,
            pltpu.SemaphoreType.DMA, pltpu.SemaphoreType.DMA,
        ],
        compiler_params=pltpu.CompilerParams(use_tc_tiling_on_sc=False,
                                             needs_layout_passes=False),
    )
    def seg_kernel(scat_hbm, gath_hbm, vals_hbm, table_hbm, out_hbm,
                   gi0, gi1, vv0, vv1, si0, si1, ro0, ro1, shared,
                   semi0, semi1, semg0, semg1, sems0, sems1):
        cid = lax.axis_index("c")
        sid = lax.axis_index("s")
        base = 0 if edge_part else cid * h
        col_ids = [lax.iota(jnp.int32, 16) + 16 * d4 for d4 in range(4)]
        gis, vvs, sis, ros = (gi0, gi1), (vv0, vv1), (si0, si1), (ro0, ro1)
        semis, semgs, semss = (semi0, semi1), (semg0, semg1), (sems0, sems1)

        def zrow(r, _):
            ev = lax.broadcast(r, (16,))
            for d4 in range(4):
                plsc.store_scatter(ro0, [ev, col_ids[d4]],
                                   jnp.zeros((16,), jnp.float32))
            return _
        lax.fori_loop(0, CH, zrow, None)

        def zshared(i, _):
            s = sid + i * NS

            @pl.when(s < zslices)
            def _():
                pltpu.sync_copy(ro0, shared.at[pl.ds(s * 128, 128)])
            return _
        lax.fori_loop(0, (zslices + NS - 1) // NS, zshared, None)
        plsc.subcore_barrier()

        eoff = ((cid * NS + sid) if edge_part else sid) * epp

        def issue_idx(j, b):
            off = eoff + j * CH
            pltpu.async_copy(gath_hbm.at[pl.ds(off, CH)], gis[b], semis[b])
            pltpu.async_copy(vals_hbm.at[pl.ds(off, CH)], vvs[b], semis[b])
            pltpu.async_copy(scat_hbm.at[pl.ds(off, CH)], sis[b], semis[b])

        def drain_idx(b):
            pltpu.make_async_copy(gath_hbm.at[pl.ds(0, CH)], gis[b], semis[b]).wait()
            pltpu.make_async_copy(vals_hbm.at[pl.ds(0, CH)], vvs[b], semis[b]).wait()
            pltpu.make_async_copy(scat_hbm.at[pl.ds(0, CH)], sis[b], semis[b]).wait()

        def drain_scat(b):
            pltpu.make_async_copy(ros[b], shared.at[sis[b]], semss[b]).wait()

        def process(b):
            pltpu.make_async_copy(table_hbm.at[gis[b]], ros[b], semgs[b]).wait()

            def scale(e, _):
                ev = lax.broadcast(e, (16,))
                vs = plsc.load_gather(vvs[b], [ev])
                for d4 in range(4):
                    x = plsc.load_gather(ros[b], [ev, col_ids[d4]])
                    plsc.store_scatter(ros[b], [ev, col_ids[d4]], x * vs)
                return _
            lax.fori_loop(0, CH, scale, None)
            for g in range(CH // 16):
                loc = sis[b][pl.ds(g * 16, 16)] - base
                ok = (loc >= 0) & (loc < h)
                trash = col_ids[0] + (h + g * 16)
                sis[b][pl.ds(g * 16, 16)] = jnp.where(ok, loc, trash)
            pltpu.async_copy(ros[b], shared.at[sis[b]], semss[b], add=True)

        issue_idx(0, 0)
        issue_idx(1, 1)

        def pair(p, _):
            drain_idx(0)

            @pl.when(p > 0)
            def _():
                drain_scat(0)
            pltpu.async_copy(table_hbm.at[gis[0]], ros[0], semgs[0])
            drain_idx(1)

            @pl.when(p > 0)
            def _():
                drain_scat(1)
            pltpu.async_copy(table_hbm.at[gis[1]], ros[1], semgs[1])
            process(0)

            @pl.when(2 * p + 2 < cps)
            def _():
                issue_idx(2 * p + 2, 0)
            process(1)

            @pl.when(2 * p + 3 < cps)
            def _():
                issue_idx(2 * p + 3, 1)
            return _
        lax.fori_loop(0, cps // 2, pair, None)
        drain_scat(0)
        drain_scat(1)
        plsc.subcore_barrier()
        if edge_part:
            pltpu.sync_copy(shared.at[pl.ds(sid * rps, rps)],
                            out_hbm.at[cid, pl.ds(sid * rps, rps)])
        else:
            pltpu.sync_copy(shared.at[pl.ds(sid * rps, rps)],
                            out_hbm.at[pl.ds(cid * h + sid * rps, rps)])

    return seg_kernel


@functools.lru_cache(maxsize=None)
def _make_front(p_rui, p_rgu, p_rgi, h_u, h_i, h_g, n_mem_pad):
    mesh = plsc.VectorSubcoreMesh(core_axis_name="c", subcore_axis_name="s")
    h_max = max(h_u, h_i, h_g)
    bpw = n_mem_pad // (NC * NS)

    out_types = (
        jax.ShapeDtypeStruct((2 * h_u, D), jnp.float32),
        jax.ShapeDtypeStruct((2 * h_u, D), jnp.float32),
        jax.ShapeDtypeStruct((NC, h_i, D), jnp.float32),
        jax.ShapeDtypeStruct((NC, h_i, D), jnp.float32),
        jax.ShapeDtypeStruct((NC, h_g, D), jnp.float32),
        jax.ShapeDtypeStruct((NC, h_g, D), jnp.float32),
        jax.ShapeDtypeStruct((n_mem_pad, D), jnp.float32),
    )

    @functools.partial(
        pl.kernel,
        out_type=out_types,
        mesh=mesh,
        scratch_types=[
            pltpu.VMEM((CH,), jnp.int32), pltpu.VMEM((CH,), jnp.int32),
            pltpu.VMEM((CH,), jnp.float32), pltpu.VMEM((CH,), jnp.float32),
            pltpu.VMEM((CH,), jnp.int32), pltpu.VMEM((CH,), jnp.int32),
            pltpu.VMEM((CH, D), jnp.float32), pltpu.VMEM((CH, D), jnp.float32),
            pltpu.VMEM((CH, D), jnp.float32),
            pltpu.VMEM((bpw,), jnp.int32),
            pltpu.VMEM_SHARED((h_max + 128, D), jnp.float32),
            pltpu.SemaphoreType.DMA, pltpu.SemaphoreType.DMA,
            pltpu.SemaphoreType.DMA, pltpu.SemaphoreType.DMA,
            pltpu.SemaphoreType.DMA, pltpu.SemaphoreType.DMA,
        ],
        compiler_params=pltpu.CompilerParams(use_tc_tiling_on_sc=False,
                                             needs_layout_passes=False),
    )
    def front_kernel(rui_r, rui_c, rui_v, rgu_r, rgu_c, rgu_v,
                     rgi_r, rgi_c, rgi_v, item_emb, user_emb, group_emb,
                     mem_idx_hbm,
                     o_rui_ei, o_rgu_t_eg, o_rui_t_eu, o_rgi_t_eg,
                     o_rgi_ei, o_rgu_eu, o_me,
                     gi0, gi1, vv0, vv1, si0, si1, ro0, ro1, zbuf,
                     midx, shared,
                     semi0, semi1, semg0, semg1, sems0, sems1):
        cid = lax.axis_index("c")
        sid = lax.axis_index("s")
        col_ids = [lax.iota(jnp.int32, 16) + 16 * d4 for d4 in range(4)]
        gis, vvs, sis, ros = (gi0, gi1), (vv0, vv1), (si0, si1), (ro0, ro1)
        semis, semgs, semss = (semi0, semi1), (semg0, semg1), (sems0, sems1)

        def zrow(r, _):
            ev = lax.broadcast(r, (16,))
            for d4 in range(4):
                plsc.store_scatter(zbuf, [ev, col_ids[d4]],
                                   jnp.zeros((16,), jnp.float32))
            return _
        lax.fori_loop(0, CH, zrow, None)

        mbase = (cid * NS + sid) * bpw
        pltpu.sync_copy(mem_idx_hbm.at[pl.ds(mbase, bpw)], midx)
        pltpu.async_copy(user_emb.at[midx], ro0.at[pl.ds(0, bpw)], semg0).wait()
        pltpu.sync_copy(ro0.at[pl.ds(0, bpw)], o_me.at[pl.ds(mbase, bpw)])

        def seg_phase(scat_hbm, gath_hbm, vals_hbm, table_hbm, out_hbm,
                      h, nnz_pad, edge_part):
            epp = nnz_pad // (NC * NS if edge_part else NS)
            cps = epp // CH
            rps = h // NS
            zslices = (h + 128) // 128
            base = 0 if edge_part else cid * h

            def zsh(i, _):
                s = sid + i * NS

                @pl.when(s < zslices)
                def _():
                    pltpu.sync_copy(zbuf, shared.at[pl.ds(s * 128, 128)])
                return _
            lax.fori_loop(0, (zslices + NS - 1) // NS, zsh, None)
            plsc.subcore_barrier()

            eoff = ((cid * NS + sid) if edge_part else sid) * epp

            def issue_idx(j, b):
                off = eoff + j * CH
                pltpu.async_copy(gath_hbm.at[pl.ds(off, CH)], gis[b], semis[b])
                pltpu.async_copy(vals_hbm.at[pl.ds(off, CH)], vvs[b], semis[b])
                pltpu.async_copy(scat_hbm.at[pl.ds(off, CH)], sis[b], semis[b])

            def drain_idx(b):
                pltpu.make_async_copy(gath_hbm.at[pl.ds(0, CH)], gis[b], semis[b]).wait()
                pltpu.make_async_copy(vals_hbm.at[pl.ds(0, CH)], vvs[b], semis[b]).wait()
                pltpu.make_async_copy(scat_hbm.at[pl.ds(0, CH)], sis[b], semis[b]).wait()

            def drain_scat(b):
                pltpu.make_async_copy(ros[b], shared.at[sis[b]], semss[b]).wait()

            def process(b):
                pltpu.make_async_copy(table_hbm.at[gis[b]], ros[b], semgs[b]).wait()

                def scale(q, _):
                    for u in range(4):
                        e = q * 4 + u
                        ev = lax.broadcast(e, (16,))
                        vs = plsc.load_gather(vvs[b], [ev])
                        for d4 in range(4):
                            x = plsc.load_gather(ros[b], [ev, col_ids[d4]])
                            plsc.store_scatter(ros[b], [ev, col_ids[d4]], x * vs)
                    return _
                lax.fori_loop(0, CH // 4, scale, None)
                for g in range(CH // 16):
                    loc = sis[b][pl.ds(g * 16, 16)] - base
                    ok = (loc >= 0) & (loc < h)
                    trash = col_ids[0] + (h + g * 16)
                    sis[b][pl.ds(g * 16, 16)] = jnp.where(ok, loc, trash)
                pltpu.async_copy(ros[b], shared.at[sis[b]], semss[b], add=True)

            issue_idx(0, 0)
            issue_idx(1, 1)

            def pair(p, _):
                drain_idx(0)

                @pl.when(p > 0)
                def _():
                    drain_scat(0)
                pltpu.async_copy(table_hbm.at[gis[0]], ros[0], semgs[0])
                drain_idx(1)

                @pl.when(p > 0)
                def _():
                    drain_scat(1)
                pltpu.async_copy(table_hbm.at[gis[1]], ros[1], semgs[1])
                process(0)

                @pl.when(2 * p + 2 < cps)
                def _():
                    issue_idx(2 * p + 2, 0)
                process(1)

                @pl.when(2 * p + 3 < cps)
                def _():
                    issue_idx(2 * p + 3, 1)
                return _
            lax.fori_loop(0, cps // 2, pair, None)
            drain_scat(0)
            drain_scat(1)
            plsc.subcore_barrier()
            if edge_part:
                pltpu.sync_copy(shared.at[pl.ds(sid * rps, rps)],
                                out_hbm.at[cid, pl.ds(sid * rps, rps)])
            else:
                pltpu.sync_copy(shared.at[pl.ds(sid * rps, rps)],
                                out_hbm.at[pl.ds(cid * h + sid * rps, rps)])
            plsc.subcore_barrier()

        seg_phase(rui_r, rui_c, rui_v, item_emb, o_rui_ei, h_u, p_rui, False)
        seg_phase(rgu_c, rgu_r, rgu_v, group_emb, o_rgu_t_eg, h_u, p_rgu, False)
        seg_phase(rui_c, rui_r, rui_v, user_emb, o_rui_t_eu, h_i, p_rui, True)
        seg_phase(rgi_c, rgi_r, rgi_v, group_emb, o_rgi_t_eg, h_i, p_rgi, True)
        seg_phase(rgi_r, rgi_c, rgi_v, item_emb, o_rgi_ei, h_g, p_rgi, True)
        seg_phase(rgu_r, rgu_c, rgu_v, user_emb, o_rgu_eu, h_g, p_rgu, True)

    return front_kernel


def _pad_edges(rows, cols, vals, mult, n_r, n_c):
    nnz = rows.shape[0]
    nnz_pad = _ceil_to(nnz, mult)
    pad = nnz_pad - nnz
    if pad:
        pad_ids = jnp.arange(pad, dtype=jnp.int32)
        rows = jnp.concatenate([rows, pad_ids % n_r])
        cols = jnp.concatenate([cols, pad_ids % n_c])
        vals = jnp.concatenate([vals, jnp.zeros((pad,), jnp.float32)])
    return rows, cols, vals, nnz_pad


def _segsum(scat_idx, gath_idx, vals, table, n_rows):
    edge_part = n_rows <= 12288
    scat_idx, gath_idx, vals, nnz_pad = _pad_edges(
        scat_idx, gath_idx, vals, 2 * CH * (NC * NS if edge_part else NS),
        n_rows, table.shape[0])
    if edge_part:
        h = _ceil_to(n_rows, 128)
        out = _make_segsum(nnz_pad, h, True)(scat_idx, gath_idx, vals, table)
        return [out[0, :n_rows], out[1, :n_rows]]
    h = _ceil_to(_ceil_to(n_rows, 2) // 2, 128)
    out = _make_segsum(nnz_pad, h, False)(scat_idx, gath_idx, vals, table)
    return [out[:n_rows]]


IB = 1024


def _att_colsum(item_pad, me_pad, n_items):
    ip, _ = item_pad.shape
    mp, _ = me_pad.shape
    nb = ip // IB

    def body(x_ref, me_ref, s_ref):
        @pl.when(pl.program_id(0) == 0)
        def _():
            s_ref[...] = jnp.zeros_like(s_ref)
        e = lax.dot_general(x_ref[...], me_ref[...], (((1,), (1,)), ((), ())),
                            preferred_element_type=jnp.float32)
        rid = pl.program_id(0) * IB + lax.broadcasted_iota(jnp.int32, (IB, 1), 0)
        contrib = jnp.where(rid < n_items, jnp.exp(e), 0.0)
        s_ref[...] += jnp.sum(contrib, axis=0, keepdims=True)

    return pl.pallas_call(
        body,
        grid=(nb,),
        in_specs=[pl.BlockSpec((IB, D), lambda i: (i, 0)),
                  pl.BlockSpec((mp, D), lambda i: (0, 0))],
        out_specs=pl.BlockSpec((1, mp), lambda i: (0, 0)),
        out_shape=jax.ShapeDtypeStruct((1, mp), jnp.float32),
    )(item_pad, me_pad)


def _att_apply(item_pad, me_pad, colsum):
    ip, _ = item_pad.shape
    mp, _ = me_pad.shape
    nb = ip // IB

    def body(x_ref, me_ref, s_ref, o_ref):
        x = x_ref[...]
        e = lax.dot_general(x, me_ref[...], (((1,), (1,)), ((), ())),
                            preferred_element_type=jnp.float32)
        w = jnp.exp(e) / s_ref[...]
        att = lax.dot_general(w, me_ref[...], (((1,), (0,)), ((), ())),
                              preferred_element_type=jnp.float32)
        o_ref[...] = att * x

    return pl.pallas_call(
        body,
        grid=(nb,),
        in_specs=[pl.BlockSpec((IB, D), lambda i: (i, 0)),
                  pl.BlockSpec((mp, D), lambda i: (0, 0)),
                  pl.BlockSpec((1, mp), lambda i: (0, 0))],
        out_specs=pl.BlockSpec((IB, D), lambda i: (i, 0)),
        out_shape=jax.ShapeDtypeStruct((ip, D), jnp.float32),
    )(item_pad, me_pad, colsum)


RB = 512


def _finish(acc):
    y = jnp.where(acc >= 0, acc, 0.01 * acc)
    nrm = jnp.sqrt(jnp.sum(y * y, axis=1, keepdims=True))
    return y / jnp.maximum(nrm, 1e-12)


def _dlin(x, w_ref, k):
    return lax.dot_general(x, w_ref[k], (((1,), (1,)), ((), ())),
                           preferred_element_type=jnp.float32)


def _combine(base, part_lists, W, bias, n_rows, group_pattern):
    np_ = _ceil_to(n_rows, RB)

    def padr(z):
        return jnp.pad(z, ((0, np_ - n_rows), (0, 0)))

    counts = [len(pl_) for pl_ in part_lists]

    def body(*refs):
        x = refs[0][...]
        pos = 1
        aggs = []
        for c in counts:
            agg = refs[pos][...]
            for r in refs[pos + 1:pos + c]:
                agg = agg + r[...]
            aggs.append(agg)
            pos += c
        w_ref, bias_ref, o_ref = refs[pos], refs[pos + 1], refs[pos + 2]
        acc = jnp.sum(bias_ref[...], axis=0, keepdims=True)
        if group_pattern:
            a, b, c = aggs
            acc = (acc + _dlin(x, w_ref, 0) + _dlin(a, w_ref, 1)
                   + _dlin(b * x, w_ref, 2) + _dlin(a * x, w_ref, 3)
                   + _dlin(c, w_ref, 4))
        else:
            a, b = aggs
            acc = (acc + _dlin(x, w_ref, 0) + _dlin(a, w_ref, 1)
                   + _dlin(a * x, w_ref, 2) + _dlin(b * x, w_ref, 3)
                   + _dlin(b, w_ref, 4))
        o_ref[...] = _finish(acc)

    flat_parts = [p for pl_ in part_lists for p in pl_]
    n_data = 1 + len(flat_parts)
    out = pl.pallas_call(
        body,
        grid=(np_ // RB,),
        in_specs=[pl.BlockSpec((RB, D), lambda i: (i, 0))] * n_data
        + [pl.BlockSpec((5, D, D), lambda i: (0, 0, 0)),
           pl.BlockSpec((5, D), lambda i: (0, 0))],
        out_specs=pl.BlockSpec((RB, D), lambda i: (i, 0)),
        out_shape=jax.ShapeDtypeStruct((np_, D), jnp.float32),
    )(padr(base), *[padr(p) for p in flat_parts], W, bias)
    return out[:n_rows]


def kernel(group_embedding, user_embedding, item_embedding, members,
           rui_rows, rui_cols, rui_vals, rgu_rows, rgu_cols, rgu_vals,
           rgi_rows, rgi_cols, rgi_vals, Wg, bg, Wu, bu, Wi, bi):
    G, U, I = group_embedding.shape[0], user_embedding.shape[0], item_embedding.shape[0]

    h_u = _ceil_to(_ceil_to(U, 2) // 2, 128)
    h_i = _ceil_to(I, 128)
    h_g = _ceil_to(G, 128)
    rui_r, rui_c, rui_v, p_rui = _pad_edges(rui_rows, rui_cols, rui_vals, 2 * CH * NC * NS, U, I)
    rgu_r, rgu_c, rgu_v, p_rgu = _pad_edges(rgu_rows, rgu_cols, rgu_vals, 2 * CH * NC * NS, G, U)
    rgi_r, rgi_c, rgi_v, p_rgi = _pad_edges(rgi_rows, rgi_cols, rgi_vals, 2 * CH * NC * NS, G, I)
    mflat = members.reshape(-1).astype(jnp.int32)
    n_mem = mflat.shape[0]
    mp = _ceil_to(n_mem, 8 * NC * NS)
    if mp != n_mem:
        mflat = jnp.concatenate([mflat, jnp.zeros((mp - n_mem,), jnp.int32)])

    front = _make_front(p_rui, p_rgu, p_rgi, h_u, h_i, h_g, mp)
    (rui_ei_f, rgu_t_eg_f, rui_t_eu_p, rgi_t_eg_p, rgi_ei_p, rgu_eu_p, me) = front(
        rui_r, rui_c, rui_v, rgu_r, rgu_c, rgu_v, rgi_r, rgi_c, rgi_v,
        item_embedding, user_embedding, group_embedding, mflat)
    rui_ei = [rui_ei_f[:U]]
    rgu_t_eg = [rgu_t_eg_f[:U]]
    rui_t_eu = [rui_t_eu_p[0, :I], rui_t_eu_p[1, :I]]
    rgi_t_eg = [rgi_t_eg_p[0, :I], rgi_t_eg_p[1, :I]]
    rgi_ei = [rgi_ei_p[0, :G], rgi_ei_p[1, :G]]
    rgu_eu = [rgu_eu_p[0, :G], rgu_eu_p[1, :G]]
    me_pad = jnp.where(
        (jnp.arange(mp) < n_mem)[:, None], me, 0.0)
    ip = _ceil_to(I, IB)
    item_pad = jnp.pad(item_embedding, ((0, ip - I), (0, 0)))
    colsum = _att_colsum(item_pad, me_pad, I)
    attentive = _att_apply(item_pad, me_pad, colsum)

    atten_g = _segsum(rgi_rows, rgi_cols, rgi_vals, attentive, G)

    nu = _combine(user_embedding, [rui_ei, rgu_t_eg], Wu, bu, U, False)
    ni = _combine(item_embedding, [rui_t_eu, rgi_t_eg], Wi, bi, I, False)
    ng = _combine(group_embedding, [rgi_ei, rgu_eu, atten_g], Wg, bg, G, True)
    return ng, nu, ni


# speedup vs baseline: 1.7068x; 1.2370x over previous
import functools
import jax
import jax.numpy as jnp
from jax import lax
from jax.experimental import pallas as pl
from jax.experimental.pallas import tpu as pltpu
from jax.experimental.pallas import tpu_sc as plsc

D = 64
NC = 2
NS = 16
CH = 128


def _ceil_to(x, m):
    return (x + m - 1) // m * m


def _lane_bcast(v16, e):
    idx = jnp.full((16, 1), e, dtype=jnp.int32)
    dnums = lax.GatherDimensionNumbers(
        offset_dims=(), collapsed_slice_dims=(0,), start_index_map=(0,))
    return lax.gather(v16, idx, dnums, (1,),
                      mode=lax.GatherScatterMode.PROMISE_IN_BOUNDS)


@functools.lru_cache(maxsize=None)
def _make_segsum(nnz_pad, h, edge_part):
    nworkers = NC * NS if edge_part else NS
    epp = nnz_pad // nworkers
    cps = epp // CH
    rps = h // NS
    zslices = (h + 128) // 128
    mesh = plsc.VectorSubcoreMesh(core_axis_name="c", subcore_axis_name="s")
    out_sds = (jax.ShapeDtypeStruct((NC, h, D), jnp.float32) if edge_part
               else jax.ShapeDtypeStruct((2 * h, D), jnp.float32))

    @functools.partial(
        pl.kernel,
        out_type=out_sds,
        mesh=mesh,
        scratch_types=[
            pltpu.VMEM((CH,), jnp.int32), pltpu.VMEM((CH,), jnp.int32),
            pltpu.VMEM((CH,), jnp.float32), pltpu.VMEM((CH,), jnp.float32),
            pltpu.VMEM((CH,), jnp.int32), pltpu.VMEM((CH,), jnp.int32),
            pltpu.VMEM((CH, D), jnp.float32), pltpu.VMEM((CH, D), jnp.float32),
            pltpu.VMEM_SHARED((h + 128, D), jnp.float32),
            pltpu.SemaphoreType.DMA, pltpu.SemaphoreType.DMA,
            pltpu.SemaphoreType.DMA, pltpu.SemaphoreType.DMA,
            pltpu.SemaphoreType.DMA, pltpu.SemaphoreType.DMA,
        ],
        compiler_params=pltpu.CompilerParams(use_tc_tiling_on_sc=False,
                                             needs_layout_passes=False),
    )
    def seg_kernel(scat_hbm, gath_hbm, vals_hbm, table_hbm, out_hbm,
                   gi0, gi1, vv0, vv1, si0, si1, ro0, ro1, shared,
                   semi0, semi1, semg0, semg1, sems0, sems1):
        cid = lax.axis_index("c")
        sid = lax.axis_index("s")
        base = 0 if edge_part else cid * h
        col_ids = [lax.iota(jnp.int32, 16) + 16 * d4 for d4 in range(4)]
        gis, vvs, sis, ros = (gi0, gi1), (vv0, vv1), (si0, si1), (ro0, ro1)
        semis, semgs, semss = (semi0, semi1), (semg0, semg1), (sems0, sems1)

        def zrow(r, _):
            ev = lax.broadcast(r, (16,))
            for d4 in range(4):
                plsc.store_scatter(ro0, [ev, col_ids[d4]],
                                   jnp.zeros((16,), jnp.float32))
            return _
        lax.fori_loop(0, CH, zrow, None)

        def zshared(i, _):
            s = sid + i * NS

            @pl.when(s < zslices)
            def _():
                pltpu.sync_copy(ro0, shared.at[pl.ds(s * 128, 128)])
            return _
        lax.fori_loop(0, (zslices + NS - 1) // NS, zshared, None)
        plsc.subcore_barrier()

        eoff = ((cid * NS + sid) if edge_part else sid) * epp

        def issue_idx(j, b):
            off = eoff + j * CH
            pltpu.async_copy(gath_hbm.at[pl.ds(off, CH)], gis[b], semis[b])
            pltpu.async_copy(vals_hbm.at[pl.ds(off, CH)], vvs[b], semis[b])
            pltpu.async_copy(scat_hbm.at[pl.ds(off, CH)], sis[b], semis[b])

        def drain_idx(b):
            pltpu.make_async_copy(gath_hbm.at[pl.ds(0, CH)], gis[b], semis[b]).wait()
            pltpu.make_async_copy(vals_hbm.at[pl.ds(0, CH)], vvs[b], semis[b]).wait()
            pltpu.make_async_copy(scat_hbm.at[pl.ds(0, CH)], sis[b], semis[b]).wait()

        def drain_scat(b):
            pltpu.make_async_copy(ros[b], shared.at[sis[b]], semss[b]).wait()

        def process(b):
            pltpu.make_async_copy(table_hbm.at[gis[b]], ros[b], semgs[b]).wait()

            def scale(e, _):
                ev = lax.broadcast(e, (16,))
                vs = plsc.load_gather(vvs[b], [ev])
                for d4 in range(4):
                    x = plsc.load_gather(ros[b], [ev, col_ids[d4]])
                    plsc.store_scatter(ros[b], [ev, col_ids[d4]], x * vs)
                return _
            lax.fori_loop(0, CH, scale, None)
            for g in range(CH // 16):
                loc = sis[b][pl.ds(g * 16, 16)] - base
                ok = (loc >= 0) & (loc < h)
                trash = col_ids[0] + (h + g * 16)
                sis[b][pl.ds(g * 16, 16)] = jnp.where(ok, loc, trash)
            pltpu.async_copy(ros[b], shared.at[sis[b]], semss[b], add=True)

        issue_idx(0, 0)
        issue_idx(1, 1)

        def pair(p, _):
            drain_idx(0)

            @pl.when(p > 0)
            def _():
                drain_scat(0)
            pltpu.async_copy(table_hbm.at[gis[0]], ros[0], semgs[0])
            drain_idx(1)

            @pl.when(p > 0)
            def _():
                drain_scat(1)
            pltpu.async_copy(table_hbm.at[gis[1]], ros[1], semgs[1])
            process(0)

            @pl.when(2 * p + 2 < cps)
            def _():
                issue_idx(2 * p + 2, 0)
            process(1)

            @pl.when(2 * p + 3 < cps)
            def _():
                issue_idx(2 * p + 3, 1)
            return _
        lax.fori_loop(0, cps // 2, pair, None)
        drain_scat(0)
        drain_scat(1)
        plsc.subcore_barrier()
        if edge_part:
            pltpu.sync_copy(shared.at[pl.ds(sid * rps, rps)],
                            out_hbm.at[cid, pl.ds(sid * rps, rps)])
        else:
            pltpu.sync_copy(shared.at[pl.ds(sid * rps, rps)],
                            out_hbm.at[pl.ds(cid * h + sid * rps, rps)])

    return seg_kernel


@functools.lru_cache(maxsize=None)
def _make_front(p_rui, p_rgu, p_rgi, h_u, h_i, h_g, n_mem_pad):
    mesh = plsc.VectorSubcoreMesh(core_axis_name="c", subcore_axis_name="s")
    h_max = max(h_u, h_i, h_g)
    bpw = n_mem_pad // (NC * NS)

    out_types = (
        jax.ShapeDtypeStruct((2 * h_u, D), jnp.float32),
        jax.ShapeDtypeStruct((2 * h_u, D), jnp.float32),
        jax.ShapeDtypeStruct((NC, h_i, D), jnp.float32),
        jax.ShapeDtypeStruct((NC, h_i, D), jnp.float32),
        jax.ShapeDtypeStruct((NC, h_g, D), jnp.float32),
        jax.ShapeDtypeStruct((NC, h_g, D), jnp.float32),
        jax.ShapeDtypeStruct((n_mem_pad, D), jnp.float32),
    )

    @functools.partial(
        pl.kernel,
        out_type=out_types,
        mesh=mesh,
        scratch_types=[
            pltpu.VMEM((CH,), jnp.int32), pltpu.VMEM((CH,), jnp.int32),
            pltpu.VMEM((CH,), jnp.float32), pltpu.VMEM((CH,), jnp.float32),
            pltpu.VMEM((CH,), jnp.int32), pltpu.VMEM((CH,), jnp.int32),
            pltpu.VMEM((CH, D), jnp.float32), pltpu.VMEM((CH, D), jnp.float32),
            pltpu.VMEM((CH, D), jnp.float32),
            pltpu.VMEM((bpw,), jnp.int32),
            pltpu.VMEM_SHARED((h_max + 128, D), jnp.float32),
            pltpu.SemaphoreType.DMA, pltpu.SemaphoreType.DMA,
            pltpu.SemaphoreType.DMA, pltpu.SemaphoreType.DMA,
            pltpu.SemaphoreType.DMA, pltpu.SemaphoreType.DMA,
        ],
        compiler_params=pltpu.CompilerParams(use_tc_tiling_on_sc=False,
                                             needs_layout_passes=False),
    )
    def front_kernel(rui_r, rui_c, rui_v, rgu_r, rgu_c, rgu_v,
                     rgi_r, rgi_c, rgi_v, item_emb, user_emb, group_emb,
                     mem_idx_hbm,
                     o_rui_ei, o_rgu_t_eg, o_rui_t_eu, o_rgi_t_eg,
                     o_rgi_ei, o_rgu_eu, o_me,
                     gi0, gi1, vv0, vv1, si0, si1, ro0, ro1, zbuf,
                     midx, shared,
                     semi0, semi1, semg0, semg1, sems0, sems1):
        cid = lax.axis_index("c")
        sid = lax.axis_index("s")
        col_ids = [lax.iota(jnp.int32, 16) + 16 * d4 for d4 in range(4)]
        gis, vvs, sis, ros = (gi0, gi1), (vv0, vv1), (si0, si1), (ro0, ro1)
        semis, semgs, semss = (semi0, semi1), (semg0, semg1), (sems0, sems1)

        def zrow(r, _):
            ev = lax.broadcast(r, (16,))
            for d4 in range(4):
                plsc.store_scatter(zbuf, [ev, col_ids[d4]],
                                   jnp.zeros((16,), jnp.float32))
            return _
        lax.fori_loop(0, CH, zrow, None)

        mbase = (cid * NS + sid) * bpw
        pltpu.sync_copy(mem_idx_hbm.at[pl.ds(mbase, bpw)], midx)
        pltpu.async_copy(user_emb.at[midx], ro0.at[pl.ds(0, bpw)], semg0).wait()
        pltpu.sync_copy(ro0.at[pl.ds(0, bpw)], o_me.at[pl.ds(mbase, bpw)])

        def seg_phase(scat_hbm, gath_hbm, vals_hbm, table_hbm, out_hbm,
                      h, nnz_pad, edge_part):
            epp = nnz_pad // (NC * NS if edge_part else NS)
            cps = epp // CH
            rps = h // NS
            zslices = (h + 128) // 128
            base = 0 if edge_part else cid * h

            def zsh(i, _):
                s = sid + i * NS

                @pl.when(s < zslices)
                def _():
                    pltpu.sync_copy(zbuf, shared.at[pl.ds(s * 128, 128)])
                return _
            lax.fori_loop(0, (zslices + NS - 1) // NS, zsh, None)
            plsc.subcore_barrier()

            eoff = ((cid * NS + sid) if edge_part else sid) * epp

            def issue_idx(j, b):
                off = eoff + j * CH
                pltpu.async_copy(gath_hbm.at[pl.ds(off, CH)], gis[b], semis[b])
                pltpu.async_copy(vals_hbm.at[pl.ds(off, CH)], vvs[b], semis[b])
                pltpu.async_copy(scat_hbm.at[pl.ds(off, CH)], sis[b], semis[b])

            def drain_idx(b):
                pltpu.make_async_copy(gath_hbm.at[pl.ds(0, CH)], gis[b], semis[b]).wait()
                pltpu.make_async_copy(vals_hbm.at[pl.ds(0, CH)], vvs[b], semis[b]).wait()
                pltpu.make_async_copy(scat_hbm.at[pl.ds(0, CH)], sis[b], semis[b]).wait()

            def drain_scat(b):
                pltpu.make_async_copy(ros[b], shared.at[sis[b]], semss[b]).wait()

            def process(b):
                pltpu.make_async_copy(table_hbm.at[gis[b]], ros[b], semgs[b]).wait()

                def scale(g, _):
                    v16 = vvs[b][pl.ds(g * 16, 16)]
                    for e in range(16):
                        vs = _lane_bcast(v16, e)
                        row = g * 16 + e
                        for d4 in range(4):
                            sl = pl.ds(d4 * 16, 16)
                            ros[b][row, sl] = ros[b][row, sl] * vs
                    return _
                lax.fori_loop(0, CH // 16, scale, None)
                for g in range(CH // 16):
                    loc = sis[b][pl.ds(g * 16, 16)] - base
                    ok = (loc >= 0) & (loc < h)
                    trash = col_ids[0] + (h + g * 16)
                    sis[b][pl.ds(g * 16, 16)] = jnp.where(ok, loc, trash)
                pltpu.async_copy(ros[b], shared.at[sis[b]], semss[b], add=True)

            issue_idx(0, 0)
            issue_idx(1, 1)

            def pair(p, _):
                drain_idx(0)

                @pl.when(p > 0)
                def _():
                    drain_scat(0)
                pltpu.async_copy(table_hbm.at[gis[0]], ros[0], semgs[0])
                drain_idx(1)

                @pl.when(p > 0)
                def _():
                    drain_scat(1)
                pltpu.async_copy(table_hbm.at[gis[1]], ros[1], semgs[1])
                process(0)

                @pl.when(2 * p + 2 < cps)
                def _():
                    issue_idx(2 * p + 2, 0)
                process(1)

                @pl.when(2 * p + 3 < cps)
                def _():
                    issue_idx(2 * p + 3, 1)
                return _
            lax.fori_loop(0, cps // 2, pair, None)
            drain_scat(0)
            drain_scat(1)
            plsc.subcore_barrier()
            if edge_part:
                pltpu.sync_copy(shared.at[pl.ds(sid * rps, rps)],
                                out_hbm.at[cid, pl.ds(sid * rps, rps)])
            else:
                pltpu.sync_copy(shared.at[pl.ds(sid * rps, rps)],
                                out_hbm.at[pl.ds(cid * h + sid * rps, rps)])
            plsc.subcore_barrier()

        seg_phase(rui_r, rui_c, rui_v, item_emb, o_rui_ei, h_u, p_rui, False)
        seg_phase(rgu_c, rgu_r, rgu_v, group_emb, o_rgu_t_eg, h_u, p_rgu, False)
        seg_phase(rui_c, rui_r, rui_v, user_emb, o_rui_t_eu, h_i, p_rui, True)
        seg_phase(rgi_c, rgi_r, rgi_v, group_emb, o_rgi_t_eg, h_i, p_rgi, True)
        seg_phase(rgi_r, rgi_c, rgi_v, item_emb, o_rgi_ei, h_g, p_rgi, True)
        seg_phase(rgu_r, rgu_c, rgu_v, user_emb, o_rgu_eu, h_g, p_rgu, True)

    return front_kernel


def _pad_edges(rows, cols, vals, mult, n_r, n_c):
    nnz = rows.shape[0]
    nnz_pad = _ceil_to(nnz, mult)
    pad = nnz_pad - nnz
    if pad:
        pad_ids = jnp.arange(pad, dtype=jnp.int32)
        rows = jnp.concatenate([rows, pad_ids % n_r])
        cols = jnp.concatenate([cols, pad_ids % n_c])
        vals = jnp.concatenate([vals, jnp.zeros((pad,), jnp.float32)])
    return rows, cols, vals, nnz_pad


def _segsum(scat_idx, gath_idx, vals, table, n_rows):
    edge_part = n_rows <= 12288
    scat_idx, gath_idx, vals, nnz_pad = _pad_edges(
        scat_idx, gath_idx, vals, 2 * CH * (NC * NS if edge_part else NS),
        n_rows, table.shape[0])
    if edge_part:
        h = _ceil_to(n_rows, 128)
        out = _make_segsum(nnz_pad, h, True)(scat_idx, gath_idx, vals, table)
        return [out[0, :n_rows], out[1, :n_rows]]
    h = _ceil_to(_ceil_to(n_rows, 2) // 2, 128)
    out = _make_segsum(nnz_pad, h, False)(scat_idx, gath_idx, vals, table)
    return [out[:n_rows]]


IB = 1024


def _att_colsum(item_pad, me_pad, n_items):
    ip, _ = item_pad.shape
    mp, _ = me_pad.shape
    nb = ip // IB

    def body(x_ref, me_ref, s_ref):
        @pl.when(pl.program_id(0) == 0)
        def _():
            s_ref[...] = jnp.zeros_like(s_ref)
        e = lax.dot_general(x_ref[...], me_ref[...], (((1,), (1,)), ((), ())),
                            preferred_element_type=jnp.float32)
        rid = pl.program_id(0) * IB + lax.broadcasted_iota(jnp.int32, (IB, 1), 0)
        contrib = jnp.where(rid < n_items, jnp.exp(e), 0.0)
        s_ref[...] += jnp.sum(contrib, axis=0, keepdims=True)

    return pl.pallas_call(
        body,
        grid=(nb,),
        in_specs=[pl.BlockSpec((IB, D), lambda i: (i, 0)),
                  pl.BlockSpec((mp, D), lambda i: (0, 0))],
        out_specs=pl.BlockSpec((1, mp), lambda i: (0, 0)),
        out_shape=jax.ShapeDtypeStruct((1, mp), jnp.float32),
    )(item_pad, me_pad)


def _att_apply(item_pad, me_pad, colsum):
    ip, _ = item_pad.shape
    mp, _ = me_pad.shape
    nb = ip // IB

    def body(x_ref, me_ref, s_ref, o_ref):
        x = x_ref[...]
        e = lax.dot_general(x, me_ref[...], (((1,), (1,)), ((), ())),
                            preferred_element_type=jnp.float32)
        w = jnp.exp(e) / s_ref[...]
        att = lax.dot_general(w, me_ref[...], (((1,), (0,)), ((), ())),
                              preferred_element_type=jnp.float32)
        o_ref[...] = att * x

    return pl.pallas_call(
        body,
        grid=(nb,),
        in_specs=[pl.BlockSpec((IB, D), lambda i: (i, 0)),
                  pl.BlockSpec((mp, D), lambda i: (0, 0)),
                  pl.BlockSpec((1, mp), lambda i: (0, 0))],
        out_specs=pl.BlockSpec((IB, D), lambda i: (i, 0)),
        out_shape=jax.ShapeDtypeStruct((ip, D), jnp.float32),
    )(item_pad, me_pad, colsum)


RB = 512


def _finish(acc):
    y = jnp.where(acc >= 0, acc, 0.01 * acc)
    nrm = jnp.sqrt(jnp.sum(y * y, axis=1, keepdims=True))
    return y / jnp.maximum(nrm, 1e-12)


def _dlin(x, w_ref, k):
    return lax.dot_general(x, w_ref[k], (((1,), (1,)), ((), ())),
                           preferred_element_type=jnp.float32)


def _combine(base, part_lists, W, bias, n_rows, group_pattern):
    np_ = _ceil_to(n_rows, RB)

    def padr(z):
        return jnp.pad(z, ((0, np_ - n_rows), (0, 0)))

    counts = [len(pl_) for pl_ in part_lists]

    def body(*refs):
        x = refs[0][...]
        pos = 1
        aggs = []
        for c in counts:
            agg = refs[pos][...]
            for r in refs[pos + 1:pos + c]:
                agg = agg + r[...]
            aggs.append(agg)
            pos += c
        w_ref, bias_ref, o_ref = refs[pos], refs[pos + 1], refs[pos + 2]
        acc = jnp.sum(bias_ref[...], axis=0, keepdims=True)
        if group_pattern:
            a, b, c = aggs
            acc = (acc + _dlin(x, w_ref, 0) + _dlin(a, w_ref, 1)
                   + _dlin(b * x, w_ref, 2) + _dlin(a * x, w_ref, 3)
                   + _dlin(c, w_ref, 4))
        else:
            a, b = aggs
            acc = (acc + _dlin(x, w_ref, 0) + _dlin(a, w_ref, 1)
                   + _dlin(a * x, w_ref, 2) + _dlin(b * x, w_ref, 3)
                   + _dlin(b, w_ref, 4))
        o_ref[...] = _finish(acc)

    flat_parts = [p for pl_ in part_lists for p in pl_]
    n_data = 1 + len(flat_parts)
    out = pl.pallas_call(
        body,
        grid=(np_ // RB,),
        in_specs=[pl.BlockSpec((RB, D), lambda i: (i, 0))] * n_data
        + [pl.BlockSpec((5, D, D), lambda i: (0, 0, 0)),
           pl.BlockSpec((5, D), lambda i: (0, 0))],
        out_specs=pl.BlockSpec((RB, D), lambda i: (i, 0)),
        out_shape=jax.ShapeDtypeStruct((np_, D), jnp.float32),
    )(padr(base), *[padr(p) for p in flat_parts], W, bias)
    return out[:n_rows]


def kernel(group_embedding, user_embedding, item_embedding, members,
           rui_rows, rui_cols, rui_vals, rgu_rows, rgu_cols, rgu_vals,
           rgi_rows, rgi_cols, rgi_vals, Wg, bg, Wu, bu, Wi, bi):
    G, U, I = group_embedding.shape[0], user_embedding.shape[0], item_embedding.shape[0]

    h_u = _ceil_to(_ceil_to(U, 2) // 2, 128)
    h_i = _ceil_to(I, 128)
    h_g = _ceil_to(G, 128)
    rui_r, rui_c, rui_v, p_rui = _pad_edges(rui_rows, rui_cols, rui_vals, 2 * CH * NC * NS, U, I)
    rgu_r, rgu_c, rgu_v, p_rgu = _pad_edges(rgu_rows, rgu_cols, rgu_vals, 2 * CH * NC * NS, G, U)
    rgi_r, rgi_c, rgi_v, p_rgi = _pad_edges(rgi_rows, rgi_cols, rgi_vals, 2 * CH * NC * NS, G, I)
    mflat = members.reshape(-1).astype(jnp.int32)
    n_mem = mflat.shape[0]
    mp = _ceil_to(n_mem, 8 * NC * NS)
    if mp != n_mem:
        mflat = jnp.concatenate([mflat, jnp.zeros((mp - n_mem,), jnp.int32)])

    front = _make_front(p_rui, p_rgu, p_rgi, h_u, h_i, h_g, mp)
    (rui_ei_f, rgu_t_eg_f, rui_t_eu_p, rgi_t_eg_p, rgi_ei_p, rgu_eu_p, me) = front(
        rui_r, rui_c, rui_v, rgu_r, rgu_c, rgu_v, rgi_r, rgi_c, rgi_v,
        item_embedding, user_embedding, group_embedding, mflat)
    rui_ei = [rui_ei_f[:U]]
    rgu_t_eg = [rgu_t_eg_f[:U]]
    rui_t_eu = [rui_t_eu_p[0, :I], rui_t_eu_p[1, :I]]
    rgi_t_eg = [rgi_t_eg_p[0, :I], rgi_t_eg_p[1, :I]]
    rgi_ei = [rgi_ei_p[0, :G], rgi_ei_p[1, :G]]
    rgu_eu = [rgu_eu_p[0, :G], rgu_eu_p[1, :G]]
    me_pad = jnp.where(
        (jnp.arange(mp) < n_mem)[:, None], me, 0.0)
    ip = _ceil_to(I, IB)
    item_pad = jnp.pad(item_embedding, ((0, ip - I), (0, 0)))
    colsum = _att_colsum(item_pad, me_pad, I)
    attentive = _att_apply(item_pad, me_pad, colsum)

    atten_g = _segsum(rgi_rows, rgi_cols, rgi_vals, attentive, G)

    nu = _combine(user_embedding, [rui_ei, rgu_t_eg], Wu, bu, U, False)
    ni = _combine(item_embedding, [rui_t_eu, rgi_t_eg], Wi, bi, I, False)
    ng = _combine(group_embedding, [rgi_ei, rgu_eu, atten_g], Wg, bg, G, True)
    return ng, nu, ni


# speedup vs baseline: 2.6337x; 1.5431x over previous
import functools
import jax
import jax.numpy as jnp
from jax import lax
from jax.experimental import pallas as pl
from jax.experimental.pallas import tpu as pltpu
from jax.experimental.pallas import tpu_sc as plsc

D = 64
NC = 2
NS = 16
CH = 128


def _ceil_to(x, m):
    return (x + m - 1) // m * m


def _lane_bcast(v16, e):
    idx = jnp.full((16, 1), e, dtype=jnp.int32)
    dnums = lax.GatherDimensionNumbers(
        offset_dims=(), collapsed_slice_dims=(0,), start_index_map=(0,))
    return lax.gather(v16, idx, dnums, (1,),
                      mode=lax.GatherScatterMode.PROMISE_IN_BOUNDS)


@functools.lru_cache(maxsize=None)
def _make_segsum(nnz_pad, h, edge_part):
    nworkers = NC * NS if edge_part else NS
    epp = nnz_pad // nworkers
    cps = epp // CH
    rps = h // NS
    zslices = (h + 128) // 128
    mesh = plsc.VectorSubcoreMesh(core_axis_name="c", subcore_axis_name="s")
    out_sds = (jax.ShapeDtypeStruct((NC, h, D), jnp.float32) if edge_part
               else jax.ShapeDtypeStruct((2 * h, D), jnp.float32))

    @functools.partial(
        pl.kernel,
        out_type=out_sds,
        mesh=mesh,
        scratch_types=[
            pltpu.VMEM((CH,), jnp.int32), pltpu.VMEM((CH,), jnp.int32),
            pltpu.VMEM((CH,), jnp.float32), pltpu.VMEM((CH,), jnp.float32),
            pltpu.VMEM((CH,), jnp.int32), pltpu.VMEM((CH,), jnp.int32),
            pltpu.VMEM((CH, D), jnp.float32), pltpu.VMEM((CH, D), jnp.float32),
            pltpu.VMEM_SHARED((h + 128, D), jnp.float32),
            pltpu.SemaphoreType.DMA, pltpu.SemaphoreType.DMA,
            pltpu.SemaphoreType.DMA, pltpu.SemaphoreType.DMA,
            pltpu.SemaphoreType.DMA, pltpu.SemaphoreType.DMA,
        ],
        compiler_params=pltpu.CompilerParams(use_tc_tiling_on_sc=False,
                                             needs_layout_passes=False),
    )
    def seg_kernel(scat_hbm, gath_hbm, vals_hbm, table_hbm, out_hbm,
                   gi0, gi1, vv0, vv1, si0, si1, ro0, ro1, shared,
                   semi0, semi1, semg0, semg1, sems0, sems1):
        cid = lax.axis_index("c")
        sid = lax.axis_index("s")
        base = 0 if edge_part else cid * h
        col_ids = [lax.iota(jnp.int32, 16) + 16 * d4 for d4 in range(4)]
        gis, vvs, sis, ros = (gi0, gi1), (vv0, vv1), (si0, si1), (ro0, ro1)
        semis, semgs, semss = (semi0, semi1), (semg0, semg1), (sems0, sems1)

        def zrow(r, _):
            ev = lax.broadcast(r, (16,))
            for d4 in range(4):
                plsc.store_scatter(ro0, [ev, col_ids[d4]],
                                   jnp.zeros((16,), jnp.float32))
            return _
        lax.fori_loop(0, CH, zrow, None)

        def zshared(i, _):
            s = sid + i * NS

            @pl.when(s < zslices)
            def _():
                pltpu.sync_copy(ro0, shared.at[pl.ds(s * 128, 128)])
            return _
        lax.fori_loop(0, (zslices + NS - 1) // NS, zshared, None)
        plsc.subcore_barrier()

        eoff = ((cid * NS + sid) if edge_part else sid) * epp

        def issue_idx(j, b):
            off = eoff + j * CH
            pltpu.async_copy(gath_hbm.at[pl.ds(off, CH)], gis[b], semis[b])
            pltpu.async_copy(vals_hbm.at[pl.ds(off, CH)], vvs[b], semis[b])
            pltpu.async_copy(scat_hbm.at[pl.ds(off, CH)], sis[b], semis[b])

        def drain_idx(b):
            pltpu.make_async_copy(gath_hbm.at[pl.ds(0, CH)], gis[b], semis[b]).wait()
            pltpu.make_async_copy(vals_hbm.at[pl.ds(0, CH)], vvs[b], semis[b]).wait()
            pltpu.make_async_copy(scat_hbm.at[pl.ds(0, CH)], sis[b], semis[b]).wait()

        def drain_scat(b):
            pltpu.make_async_copy(ros[b], shared.at[sis[b]], semss[b]).wait()

        def process(b):
            pltpu.make_async_copy(table_hbm.at[gis[b]], ros[b], semgs[b]).wait()

            def scale(e, _):
                ev = lax.broadcast(e, (16,))
                vs = plsc.load_gather(vvs[b], [ev])
                for d4 in range(4):
                    x = plsc.load_gather(ros[b], [ev, col_ids[d4]])
                    plsc.store_scatter(ros[b], [ev, col_ids[d4]], x * vs)
                return _
            lax.fori_loop(0, CH, scale, None)
            for g in range(CH // 16):
                loc = sis[b][pl.ds(g * 16, 16)] - base
                ok = (loc >= 0) & (loc < h)
                trash = col_ids[0] + (h + g * 16)
                sis[b][pl.ds(g * 16, 16)] = jnp.where(ok, loc, trash)
            pltpu.async_copy(ros[b], shared.at[sis[b]], semss[b], add=True)

        issue_idx(0, 0)
        issue_idx(1, 1)

        def pair(p, _):
            drain_idx(0)

            @pl.when(p > 0)
            def _():
                drain_scat(0)
            pltpu.async_copy(table_hbm.at[gis[0]], ros[0], semgs[0])
            drain_idx(1)

            @pl.when(p > 0)
            def _():
                drain_scat(1)
            pltpu.async_copy(table_hbm.at[gis[1]], ros[1], semgs[1])
            process(0)

            @pl.when(2 * p + 2 < cps)
            def _():
                issue_idx(2 * p + 2, 0)
            process(1)

            @pl.when(2 * p + 3 < cps)
            def _():
                issue_idx(2 * p + 3, 1)
            return _
        lax.fori_loop(0, cps // 2, pair, None)
        drain_scat(0)
        drain_scat(1)
        plsc.subcore_barrier()
        if edge_part:
            pltpu.sync_copy(shared.at[pl.ds(sid * rps, rps)],
                            out_hbm.at[cid, pl.ds(sid * rps, rps)])
        else:
            pltpu.sync_copy(shared.at[pl.ds(sid * rps, rps)],
                            out_hbm.at[pl.ds(cid * h + sid * rps, rps)])

    return seg_kernel


@functools.lru_cache(maxsize=None)
def _make_front(p_rui, p_rgu, p_rgi, h_u, h_i, h_g, n_mem_pad):
    mesh = plsc.VectorSubcoreMesh(core_axis_name="c", subcore_axis_name="s")
    h_max = max(h_u, h_i, h_g)
    bpw = n_mem_pad // (NC * NS)

    out_types = (
        jax.ShapeDtypeStruct((2 * h_u, D), jnp.float32),
        jax.ShapeDtypeStruct((2 * h_u, D), jnp.float32),
        jax.ShapeDtypeStruct((NC, h_i, D), jnp.float32),
        jax.ShapeDtypeStruct((NC, h_i, D), jnp.float32),
        jax.ShapeDtypeStruct((NC, h_g, D), jnp.float32),
        jax.ShapeDtypeStruct((NC, h_g, D), jnp.float32),
        jax.ShapeDtypeStruct((n_mem_pad, D), jnp.float32),
    )

    @functools.partial(
        pl.kernel,
        out_type=out_types,
        mesh=mesh,
        scratch_types=[
            pltpu.VMEM((CH,), jnp.int32), pltpu.VMEM((CH,), jnp.int32),
            pltpu.VMEM((CH,), jnp.float32), pltpu.VMEM((CH,), jnp.float32),
            pltpu.VMEM((CH,), jnp.int32), pltpu.VMEM((CH,), jnp.int32),
            pltpu.VMEM((CH, D), jnp.float32), pltpu.VMEM((CH, D), jnp.float32),
            pltpu.VMEM((CH, D), jnp.float32),
            pltpu.VMEM((bpw,), jnp.int32),
            pltpu.VMEM_SHARED((h_max + 128, D), jnp.float32),
            pltpu.SemaphoreType.DMA, pltpu.SemaphoreType.DMA,
            pltpu.SemaphoreType.DMA, pltpu.SemaphoreType.DMA,
            pltpu.SemaphoreType.DMA, pltpu.SemaphoreType.DMA,
        ],
        compiler_params=pltpu.CompilerParams(use_tc_tiling_on_sc=False,
                                             needs_layout_passes=False),
    )
    def front_kernel(rui_r, rui_c, rui_v, rgu_r, rgu_c, rgu_v,
                     rgi_r, rgi_c, rgi_v, item_emb, user_emb, group_emb,
                     mem_idx_hbm,
                     o_rui_ei, o_rgu_t_eg, o_rui_t_eu, o_rgi_t_eg,
                     o_rgi_ei, o_rgu_eu, o_me,
                     gi0, gi1, vv0, vv1, si0, si1, ro0, ro1, zbuf,
                     midx, shared,
                     semi0, semi1, semg0, semg1, sems0, sems1):
        cid = lax.axis_index("c")
        sid = lax.axis_index("s")
        col_ids = [lax.iota(jnp.int32, 16) + 16 * d4 for d4 in range(4)]
        gis, vvs, sis, ros = (gi0, gi1), (vv0, vv1), (si0, si1), (ro0, ro1)
        semis, semgs, semss = (semi0, semi1), (semg0, semg1), (sems0, sems1)

        def zrow(r, _):
            ev = lax.broadcast(r, (16,))
            for d4 in range(4):
                plsc.store_scatter(zbuf, [ev, col_ids[d4]],
                                   jnp.zeros((16,), jnp.float32))
            return _
        lax.fori_loop(0, CH, zrow, None)

        mbase = (cid * NS + sid) * bpw
        pltpu.sync_copy(mem_idx_hbm.at[pl.ds(mbase, bpw)], midx)
        pltpu.async_copy(user_emb.at[midx], ro0.at[pl.ds(0, bpw)], semg0).wait()
        pltpu.sync_copy(ro0.at[pl.ds(0, bpw)], o_me.at[pl.ds(mbase, bpw)])

        def seg_phase(scat_hbm, gath_hbm, vals_hbm, table_hbm, out_hbm,
                      h, nnz_pad, edge_part):
            epp = nnz_pad // (NC * NS if edge_part else NS)
            cps = epp // CH
            rps = h // NS
            zslices = (h + 128) // 128
            base = 0 if edge_part else cid * h

            def zsh(i, _):
                s = sid + i * NS

                @pl.when(s < zslices)
                def _():
                    pltpu.sync_copy(zbuf, shared.at[pl.ds(s * 128, 128)])
                return _
            lax.fori_loop(0, (zslices + NS - 1) // NS, zsh, None)
            plsc.subcore_barrier()

            eoff = ((cid * NS + sid) if edge_part else sid) * epp

            def issue_idx(j, b):
                off = eoff + j * CH
                pltpu.async_copy(gath_hbm.at[pl.ds(off, CH)], gis[b], semis[b])
                pltpu.async_copy(vals_hbm.at[pl.ds(off, CH)], vvs[b], semis[b])
                pltpu.async_copy(scat_hbm.at[pl.ds(off, CH)], sis[b], semis[b])

            def drain_idx(b):
                pltpu.make_async_copy(gath_hbm.at[pl.ds(0, CH)], gis[b], semis[b]).wait()
                pltpu.make_async_copy(vals_hbm.at[pl.ds(0, CH)], vvs[b], semis[b]).wait()
                pltpu.make_async_copy(scat_hbm.at[pl.ds(0, CH)], sis[b], semis[b]).wait()

            def drain_scat(b):
                pltpu.make_async_copy(ros[b], shared.at[sis[b]], semss[b]).wait()

            def process(b):
                pltpu.make_async_copy(table_hbm.at[gis[b]], ros[b], semgs[b]).wait()

                @plsc.parallel_loop(0, CH, step=16)
                def _scale(i):
                    v16 = vvs[b][pl.ds(i, 16)]
                    for e in range(16):
                        vs = _lane_bcast(v16, e)
                        for d4 in range(4):
                            sl = pl.ds(d4 * 16, 16)
                            ros[b][i + e, sl] = ros[b][i + e, sl] * vs
                for g in range(CH // 16):
                    loc = sis[b][pl.ds(g * 16, 16)] - base
                    ok = (loc >= 0) & (loc < h)
                    trash = col_ids[0] + (h + g * 16)
                    sis[b][pl.ds(g * 16, 16)] = jnp.where(ok, loc, trash)
                pltpu.async_copy(ros[b], shared.at[sis[b]], semss[b], add=True)

            issue_idx(0, 0)
            issue_idx(1, 1)

            def pair(p, _):
                drain_idx(0)

                @pl.when(p > 0)
                def _():
                    drain_scat(0)
                pltpu.async_copy(table_hbm.at[gis[0]], ros[0], semgs[0])
                drain_idx(1)

                @pl.when(p > 0)
                def _():
                    drain_scat(1)
                pltpu.async_copy(table_hbm.at[gis[1]], ros[1], semgs[1])
                process(0)

                @pl.when(2 * p + 2 < cps)
                def _():
                    issue_idx(2 * p + 2, 0)
                process(1)

                @pl.when(2 * p + 3 < cps)
                def _():
                    issue_idx(2 * p + 3, 1)
                return _
            lax.fori_loop(0, cps // 2, pair, None)
            drain_scat(0)
            drain_scat(1)
            plsc.subcore_barrier()
            if edge_part:
                pltpu.sync_copy(shared.at[pl.ds(sid * rps, rps)],
                                out_hbm.at[cid, pl.ds(sid * rps, rps)])
            else:
                pltpu.sync_copy(shared.at[pl.ds(sid * rps, rps)],
                                out_hbm.at[pl.ds(cid * h + sid * rps, rps)])
            plsc.subcore_barrier()

        seg_phase(rui_r, rui_c, rui_v, item_emb, o_rui_ei, h_u, p_rui, False)
        seg_phase(rgu_c, rgu_r, rgu_v, group_emb, o_rgu_t_eg, h_u, p_rgu, False)
        seg_phase(rui_c, rui_r, rui_v, user_emb, o_rui_t_eu, h_i, p_rui, True)
        seg_phase(rgi_c, rgi_r, rgi_v, group_emb, o_rgi_t_eg, h_i, p_rgi, True)
        seg_phase(rgi_r, rgi_c, rgi_v, item_emb, o_rgi_ei, h_g, p_rgi, True)
        seg_phase(rgu_r, rgu_c, rgu_v, user_emb, o_rgu_eu, h_g, p_rgu, True)

    return front_kernel


def _pad_edges(rows, cols, vals, mult, n_r, n_c):
    nnz = rows.shape[0]
    nnz_pad = _ceil_to(nnz, mult)
    pad = nnz_pad - nnz
    if pad:
        pad_ids = jnp.arange(pad, dtype=jnp.int32)
        rows = jnp.concatenate([rows, pad_ids % n_r])
        cols = jnp.concatenate([cols, pad_ids % n_c])
        vals = jnp.concatenate([vals, jnp.zeros((pad,), jnp.float32)])
    return rows, cols, vals, nnz_pad


def _segsum(scat_idx, gath_idx, vals, table, n_rows):
    edge_part = n_rows <= 12288
    scat_idx, gath_idx, vals, nnz_pad = _pad_edges(
        scat_idx, gath_idx, vals, 2 * CH * (NC * NS if edge_part else NS),
        n_rows, table.shape[0])
    if edge_part:
        h = _ceil_to(n_rows, 128)
        out = _make_segsum(nnz_pad, h, True)(scat_idx, gath_idx, vals, table)
        return [out[0, :n_rows], out[1, :n_rows]]
    h = _ceil_to(_ceil_to(n_rows, 2) // 2, 128)
    out = _make_segsum(nnz_pad, h, False)(scat_idx, gath_idx, vals, table)
    return [out[:n_rows]]


IB = 1024


def _att_colsum(item_pad, me_pad, n_items):
    ip, _ = item_pad.shape
    mp, _ = me_pad.shape
    nb = ip // IB

    def body(x_ref, me_ref, s_ref):
        @pl.when(pl.program_id(0) == 0)
        def _():
            s_ref[...] = jnp.zeros_like(s_ref)
        e = lax.dot_general(x_ref[...], me_ref[...], (((1,), (1,)), ((), ())),
                            preferred_element_type=jnp.float32)
        rid = pl.program_id(0) * IB + lax.broadcasted_iota(jnp.int32, (IB, 1), 0)
        contrib = jnp.where(rid < n_items, jnp.exp(e), 0.0)
        s_ref[...] += jnp.sum(contrib, axis=0, keepdims=True)

    return pl.pallas_call(
        body,
        grid=(nb,),
        in_specs=[pl.BlockSpec((IB, D), lambda i: (i, 0)),
                  pl.BlockSpec((mp, D), lambda i: (0, 0))],
        out_specs=pl.BlockSpec((1, mp), lambda i: (0, 0)),
        out_shape=jax.ShapeDtypeStruct((1, mp), jnp.float32),
    )(item_pad, me_pad)


def _att_apply(item_pad, me_pad, colsum):
    ip, _ = item_pad.shape
    mp, _ = me_pad.shape
    nb = ip // IB

    def body(x_ref, me_ref, s_ref, o_ref):
        x = x_ref[...]
        e = lax.dot_general(x, me_ref[...], (((1,), (1,)), ((), ())),
                            preferred_element_type=jnp.float32)
        w = jnp.exp(e) / s_ref[...]
        att = lax.dot_general(w, me_ref[...], (((1,), (0,)), ((), ())),
                              preferred_element_type=jnp.float32)
        o_ref[...] = att * x

    return pl.pallas_call(
        body,
        grid=(nb,),
        in_specs=[pl.BlockSpec((IB, D), lambda i: (i, 0)),
                  pl.BlockSpec((mp, D), lambda i: (0, 0)),
                  pl.BlockSpec((1, mp), lambda i: (0, 0))],
        out_specs=pl.BlockSpec((IB, D), lambda i: (i, 0)),
        out_shape=jax.ShapeDtypeStruct((ip, D), jnp.float32),
    )(item_pad, me_pad, colsum)


RB = 512


def _finish(acc):
    y = jnp.where(acc >= 0, acc, 0.01 * acc)
    nrm = jnp.sqrt(jnp.sum(y * y, axis=1, keepdims=True))
    return y / jnp.maximum(nrm, 1e-12)


def _dlin(x, w_ref, k):
    return lax.dot_general(x, w_ref[k], (((1,), (1,)), ((), ())),
                           preferred_element_type=jnp.float32)


def _combine(base, part_lists, W, bias, n_rows, group_pattern):
    np_ = _ceil_to(n_rows, RB)

    def padr(z):
        return jnp.pad(z, ((0, np_ - n_rows), (0, 0)))

    counts = [len(pl_) for pl_ in part_lists]

    def body(*refs):
        x = refs[0][...]
        pos = 1
        aggs = []
        for c in counts:
            agg = refs[pos][...]
            for r in refs[pos + 1:pos + c]:
                agg = agg + r[...]
            aggs.append(agg)
            pos += c
        w_ref, bias_ref, o_ref = refs[pos], refs[pos + 1], refs[pos + 2]
        acc = jnp.sum(bias_ref[...], axis=0, keepdims=True)
        if group_pattern:
            a, b, c = aggs
            acc = (acc + _dlin(x, w_ref, 0) + _dlin(a, w_ref, 1)
                   + _dlin(b * x, w_ref, 2) + _dlin(a * x, w_ref, 3)
                   + _dlin(c, w_ref, 4))
        else:
            a, b = aggs
            acc = (acc + _dlin(x, w_ref, 0) + _dlin(a, w_ref, 1)
                   + _dlin(a * x, w_ref, 2) + _dlin(b * x, w_ref, 3)
                   + _dlin(b, w_ref, 4))
        o_ref[...] = _finish(acc)

    flat_parts = [p for pl_ in part_lists for p in pl_]
    n_data = 1 + len(flat_parts)
    out = pl.pallas_call(
        body,
        grid=(np_ // RB,),
        in_specs=[pl.BlockSpec((RB, D), lambda i: (i, 0))] * n_data
        + [pl.BlockSpec((5, D, D), lambda i: (0, 0, 0)),
           pl.BlockSpec((5, D), lambda i: (0, 0))],
        out_specs=pl.BlockSpec((RB, D), lambda i: (i, 0)),
        out_shape=jax.ShapeDtypeStruct((np_, D), jnp.float32),
    )(padr(base), *[padr(p) for p in flat_parts], W, bias)
    return out[:n_rows]


def kernel(group_embedding, user_embedding, item_embedding, members,
           rui_rows, rui_cols, rui_vals, rgu_rows, rgu_cols, rgu_vals,
           rgi_rows, rgi_cols, rgi_vals, Wg, bg, Wu, bu, Wi, bi):
    G, U, I = group_embedding.shape[0], user_embedding.shape[0], item_embedding.shape[0]

    h_u = _ceil_to(_ceil_to(U, 2) // 2, 128)
    h_i = _ceil_to(I, 128)
    h_g = _ceil_to(G, 128)
    rui_r, rui_c, rui_v, p_rui = _pad_edges(rui_rows, rui_cols, rui_vals, 2 * CH * NC * NS, U, I)
    rgu_r, rgu_c, rgu_v, p_rgu = _pad_edges(rgu_rows, rgu_cols, rgu_vals, 2 * CH * NC * NS, G, U)
    rgi_r, rgi_c, rgi_v, p_rgi = _pad_edges(rgi_rows, rgi_cols, rgi_vals, 2 * CH * NC * NS, G, I)
    mflat = members.reshape(-1).astype(jnp.int32)
    n_mem = mflat.shape[0]
    mp = _ceil_to(n_mem, 8 * NC * NS)
    if mp != n_mem:
        mflat = jnp.concatenate([mflat, jnp.zeros((mp - n_mem,), jnp.int32)])

    front = _make_front(p_rui, p_rgu, p_rgi, h_u, h_i, h_g, mp)
    (rui_ei_f, rgu_t_eg_f, rui_t_eu_p, rgi_t_eg_p, rgi_ei_p, rgu_eu_p, me) = front(
        rui_r, rui_c, rui_v, rgu_r, rgu_c, rgu_v, rgi_r, rgi_c, rgi_v,
        item_embedding, user_embedding, group_embedding, mflat)
    rui_ei = [rui_ei_f[:U]]
    rgu_t_eg = [rgu_t_eg_f[:U]]
    rui_t_eu = [rui_t_eu_p[0, :I], rui_t_eu_p[1, :I]]
    rgi_t_eg = [rgi_t_eg_p[0, :I], rgi_t_eg_p[1, :I]]
    rgi_ei = [rgi_ei_p[0, :G], rgi_ei_p[1, :G]]
    rgu_eu = [rgu_eu_p[0, :G], rgu_eu_p[1, :G]]
    me_pad = jnp.where(
        (jnp.arange(mp) < n_mem)[:, None], me, 0.0)
    ip = _ceil_to(I, IB)
    item_pad = jnp.pad(item_embedding, ((0, ip - I), (0, 0)))
    colsum = _att_colsum(item_pad, me_pad, I)
    attentive = _att_apply(item_pad, me_pad, colsum)

    atten_g = _segsum(rgi_rows, rgi_cols, rgi_vals, attentive, G)

    nu = _combine(user_embedding, [rui_ei, rgu_t_eg], Wu, bu, U, False)
    ni = _combine(item_embedding, [rui_t_eu, rgi_t_eg], Wi, bi, I, False)
    ng = _combine(group_embedding, [rgi_ei, rgu_eu, atten_g], Wg, bg, G, True)
    return ng, nu, ni


# speedup vs baseline: 2.8162x; 1.0693x over previous
import functools
import jax
import jax.numpy as jnp
from jax import lax
from jax.experimental import pallas as pl
from jax.experimental.pallas import tpu as pltpu
from jax.experimental.pallas import tpu_sc as plsc

D = 64
NC = 2
NS = 16
CH = 128


def _ceil_to(x, m):
    return (x + m - 1) // m * m


def _lane_bcast(v16, e):
    idx = jnp.full((16, 1), e, dtype=jnp.int32)
    dnums = lax.GatherDimensionNumbers(
        offset_dims=(), collapsed_slice_dims=(0,), start_index_map=(0,))
    return lax.gather(v16, idx, dnums, (1,),
                      mode=lax.GatherScatterMode.PROMISE_IN_BOUNDS)


@functools.lru_cache(maxsize=None)
def _make_segsum(nnz_pad, h, edge_part):
    nworkers = NC * NS if edge_part else NS
    epp = nnz_pad // nworkers
    cps = epp // CH
    rps = h // NS
    zslices = (h + 128) // 128
    mesh = plsc.VectorSubcoreMesh(core_axis_name="c", subcore_axis_name="s")
    out_sds = (jax.ShapeDtypeStruct((NC, h, D), jnp.float32) if edge_part
               else jax.ShapeDtypeStruct((2 * h, D), jnp.float32))

    @functools.partial(
        pl.kernel,
        out_type=out_sds,
        mesh=mesh,
        scratch_types=[
            pltpu.VMEM((CH,), jnp.int32), pltpu.VMEM((CH,), jnp.int32),
            pltpu.VMEM((CH,), jnp.float32), pltpu.VMEM((CH,), jnp.float32),
            pltpu.VMEM((CH,), jnp.int32), pltpu.VMEM((CH,), jnp.int32),
            pltpu.VMEM((CH, D), jnp.float32), pltpu.VMEM((CH, D), jnp.float32),
            pltpu.VMEM_SHARED((h + 128, D), jnp.float32),
            pltpu.SemaphoreType.DMA, pltpu.SemaphoreType.DMA,
            pltpu.SemaphoreType.DMA, pltpu.SemaphoreType.DMA,
            pltpu.SemaphoreType.DMA, pltpu.SemaphoreType.DMA,
        ],
        compiler_params=pltpu.CompilerParams(use_tc_tiling_on_sc=False,
                                             needs_layout_passes=False),
    )
    def seg_kernel(scat_hbm, gath_hbm, vals_hbm, table_hbm, out_hbm,
                   gi0, gi1, vv0, vv1, si0, si1, ro0, ro1, shared,
                   semi0, semi1, semg0, semg1, sems0, sems1):
        cid = lax.axis_index("c")
        sid = lax.axis_index("s")
        base = 0 if edge_part else cid * h
        col_ids = [lax.iota(jnp.int32, 16) + 16 * d4 for d4 in range(4)]
        gis, vvs, sis, ros = (gi0, gi1), (vv0, vv1), (si0, si1), (ro0, ro1)
        semis, semgs, semss = (semi0, semi1), (semg0, semg1), (sems0, sems1)

        def zrow(r, _):
            ev = lax.broadcast(r, (16,))
            for d4 in range(4):
                plsc.store_scatter(ro0, [ev, col_ids[d4]],
                                   jnp.zeros((16,), jnp.float32))
            return _
        lax.fori_loop(0, CH, zrow, None)

        def zshared(i, _):
            s = sid + i * NS

            @pl.when(s < zslices)
            def _():
                pltpu.sync_copy(ro0, shared.at[pl.ds(s * 128, 128)])
            return _
        lax.fori_loop(0, (zslices + NS - 1) // NS, zshared, None)
        plsc.subcore_barrier()

        eoff = ((cid * NS + sid) if edge_part else sid) * epp

        def issue_idx(j, b):
            off = eoff + j * CH
            pltpu.async_copy(gath_hbm.at[pl.ds(off, CH)], gis[b], semis[b])
            pltpu.async_copy(vals_hbm.at[pl.ds(off, CH)], vvs[b], semis[b])
            pltpu.async_copy(scat_hbm.at[pl.ds(off, CH)], sis[b], semis[b])

        def drain_idx(b):
            pltpu.make_async_copy(gath_hbm.at[pl.ds(0, CH)], gis[b], semis[b]).wait()
            pltpu.make_async_copy(vals_hbm.at[pl.ds(0, CH)], vvs[b], semis[b]).wait()
            pltpu.make_async_copy(scat_hbm.at[pl.ds(0, CH)], sis[b], semis[b]).wait()

        def drain_scat(b):
            pltpu.make_async_copy(ros[b], shared.at[sis[b]], semss[b]).wait()

        def process(b):
            pltpu.make_async_copy(table_hbm.at[gis[b]], ros[b], semgs[b]).wait()

            @plsc.parallel_loop(0, CH, step=16)
            def _scale(i):
                v16 = vvs[b][pl.ds(i, 16)]
                for e in range(16):
                    vs = _lane_bcast(v16, e)
                    for d4 in range(4):
                        sl = pl.ds(d4 * 16, 16)
                        ros[b][i + e, sl] = ros[b][i + e, sl] * vs
            for g in range(CH // 16):
                loc = sis[b][pl.ds(g * 16, 16)] - base
                ok = (loc >= 0) & (loc < h)
                trash = col_ids[0] + (h + g * 16)
                sis[b][pl.ds(g * 16, 16)] = jnp.where(ok, loc, trash)
            pltpu.async_copy(ros[b], shared.at[sis[b]], semss[b], add=True)

        issue_idx(0, 0)
        issue_idx(1, 1)

        def pair(p, _):
            drain_idx(0)

            @pl.when(p > 0)
            def _():
                drain_scat(0)
            pltpu.async_copy(table_hbm.at[gis[0]], ros[0], semgs[0])
            drain_idx(1)

            @pl.when(p > 0)
            def _():
                drain_scat(1)
            pltpu.async_copy(table_hbm.at[gis[1]], ros[1], semgs[1])
            process(0)

            @pl.when(2 * p + 2 < cps)
            def _():
                issue_idx(2 * p + 2, 0)
            process(1)

            @pl.when(2 * p + 3 < cps)
            def _():
                issue_idx(2 * p + 3, 1)
            return _
        lax.fori_loop(0, cps // 2, pair, None)
        drain_scat(0)
        drain_scat(1)
        plsc.subcore_barrier()
        if edge_part:
            pltpu.sync_copy(shared.at[pl.ds(sid * rps, rps)],
                            out_hbm.at[cid, pl.ds(sid * rps, rps)])
        else:
            pltpu.sync_copy(shared.at[pl.ds(sid * rps, rps)],
                            out_hbm.at[pl.ds(cid * h + sid * rps, rps)])

    return seg_kernel


@functools.lru_cache(maxsize=None)
def _make_front(p_rui, p_rgu, p_rgi, h_u, h_i, h_g, n_mem_pad):
    mesh = plsc.VectorSubcoreMesh(core_axis_name="c", subcore_axis_name="s")
    h_max = max(h_u, h_i, h_g)
    bpw = n_mem_pad // (NC * NS)

    out_types = (
        jax.ShapeDtypeStruct((2 * h_u, D), jnp.float32),
        jax.ShapeDtypeStruct((2 * h_u, D), jnp.float32),
        jax.ShapeDtypeStruct((NC, h_i, D), jnp.float32),
        jax.ShapeDtypeStruct((NC, h_i, D), jnp.float32),
        jax.ShapeDtypeStruct((NC, h_g, D), jnp.float32),
        jax.ShapeDtypeStruct((NC, h_g, D), jnp.float32),
        jax.ShapeDtypeStruct((n_mem_pad, D), jnp.float32),
    )

    @functools.partial(
        pl.kernel,
        out_type=out_types,
        mesh=mesh,
        scratch_types=[
            pltpu.VMEM((CH,), jnp.int32), pltpu.VMEM((CH,), jnp.int32),
            pltpu.VMEM((CH,), jnp.float32), pltpu.VMEM((CH,), jnp.float32),
            pltpu.VMEM((CH,), jnp.int32), pltpu.VMEM((CH,), jnp.int32),
            pltpu.VMEM((CH, D), jnp.float32), pltpu.VMEM((CH, D), jnp.float32),
            pltpu.VMEM((CH, D), jnp.float32),
            pltpu.VMEM((bpw,), jnp.int32),
            pltpu.VMEM_SHARED((h_max + 128, D), jnp.float32),
            pltpu.SemaphoreType.DMA, pltpu.SemaphoreType.DMA,
            pltpu.SemaphoreType.DMA, pltpu.SemaphoreType.DMA,
            pltpu.SemaphoreType.DMA, pltpu.SemaphoreType.DMA,
        ],
        compiler_params=pltpu.CompilerParams(use_tc_tiling_on_sc=False,
                                             needs_layout_passes=False),
    )
    def front_kernel(rui_r, rui_c, rui_v, rgu_r, rgu_c, rgu_v,
                     rgi_r, rgi_c, rgi_v, item_emb, user_emb, group_emb,
                     mem_idx_hbm,
                     o_rui_ei, o_rgu_t_eg, o_rui_t_eu, o_rgi_t_eg,
                     o_rgi_ei, o_rgu_eu, o_me,
                     gi0, gi1, vv0, vv1, si0, si1, ro0, ro1, zbuf,
                     midx, shared,
                     semi0, semi1, semg0, semg1, sems0, sems1):
        cid = lax.axis_index("c")
        sid = lax.axis_index("s")
        col_ids = [lax.iota(jnp.int32, 16) + 16 * d4 for d4 in range(4)]
        gis, vvs, sis, ros = (gi0, gi1), (vv0, vv1), (si0, si1), (ro0, ro1)
        semis, semgs, semss = (semi0, semi1), (semg0, semg1), (sems0, sems1)

        def zrow(r, _):
            ev = lax.broadcast(r, (16,))
            for d4 in range(4):
                plsc.store_scatter(zbuf, [ev, col_ids[d4]],
                                   jnp.zeros((16,), jnp.float32))
            return _
        lax.fori_loop(0, CH, zrow, None)

        mbase = (cid * NS + sid) * bpw
        pltpu.sync_copy(mem_idx_hbm.at[pl.ds(mbase, bpw)], midx)
        pltpu.async_copy(user_emb.at[midx], ro0.at[pl.ds(0, bpw)], semg0).wait()
        pltpu.sync_copy(ro0.at[pl.ds(0, bpw)], o_me.at[pl.ds(mbase, bpw)])

        def seg_phase(scat_hbm, gath_hbm, vals_hbm, table_hbm, out_hbm,
                      h, nnz_pad, edge_part):
            epp = nnz_pad // (NC * NS if edge_part else NS)
            cps = epp // CH
            rps = h // NS
            zslices = (h + 128) // 128
            base = 0 if edge_part else cid * h

            def zsh(i, _):
                s = sid + i * NS

                @pl.when(s < zslices)
                def _():
                    pltpu.sync_copy(zbuf, shared.at[pl.ds(s * 128, 128)])
                return _
            lax.fori_loop(0, (zslices + NS - 1) // NS, zsh, None)
            plsc.subcore_barrier()

            eoff = ((cid * NS + sid) if edge_part else sid) * epp

            def issue_idx(j, b):
                off = eoff + j * CH
                pltpu.async_copy(gath_hbm.at[pl.ds(off, CH)], gis[b], semis[b])
                pltpu.async_copy(vals_hbm.at[pl.ds(off, CH)], vvs[b], semis[b])
                pltpu.async_copy(scat_hbm.at[pl.ds(off, CH)], sis[b], semis[b])

            def drain_idx(b):
                pltpu.make_async_copy(gath_hbm.at[pl.ds(0, CH)], gis[b], semis[b]).wait()
                pltpu.make_async_copy(vals_hbm.at[pl.ds(0, CH)], vvs[b], semis[b]).wait()
                pltpu.make_async_copy(scat_hbm.at[pl.ds(0, CH)], sis[b], semis[b]).wait()

            def drain_scat(b):
                pltpu.make_async_copy(ros[b], shared.at[sis[b]], semss[b]).wait()

            def process(b):
                pltpu.make_async_copy(table_hbm.at[gis[b]], ros[b], semgs[b]).wait()

                @plsc.parallel_loop(0, CH, step=16)
                def _scale(i):
                    v16 = vvs[b][pl.ds(i, 16)]
                    for e in range(16):
                        vs = _lane_bcast(v16, e)
                        for d4 in range(4):
                            sl = pl.ds(d4 * 16, 16)
                            ros[b][i + e, sl] = ros[b][i + e, sl] * vs
                for g in range(CH // 16):
                    loc = sis[b][pl.ds(g * 16, 16)] - base
                    ok = (loc >= 0) & (loc < h)
                    trash = col_ids[0] + (h + g * 16)
                    sis[b][pl.ds(g * 16, 16)] = jnp.where(ok, loc, trash)
                pltpu.async_copy(ros[b], shared.at[sis[b]], semss[b], add=True)

            issue_idx(0, 0)
            issue_idx(1, 1)

            def pair(p, _):
                drain_idx(0)

                @pl.when(p > 0)
                def _():
                    drain_scat(0)
                pltpu.async_copy(table_hbm.at[gis[0]], ros[0], semgs[0])
                drain_idx(1)

                @pl.when(p > 0)
                def _():
                    drain_scat(1)
                pltpu.async_copy(table_hbm.at[gis[1]], ros[1], semgs[1])
                process(0)

                @pl.when(2 * p + 2 < cps)
                def _():
                    issue_idx(2 * p + 2, 0)
                process(1)

                @pl.when(2 * p + 3 < cps)
                def _():
                    issue_idx(2 * p + 3, 1)
                return _
            lax.fori_loop(0, cps // 2, pair, None)
            drain_scat(0)
            drain_scat(1)
            plsc.subcore_barrier()
            if edge_part:
                pltpu.sync_copy(shared.at[pl.ds(sid * rps, rps)],
                                out_hbm.at[cid, pl.ds(sid * rps, rps)])
            else:
                pltpu.sync_copy(shared.at[pl.ds(sid * rps, rps)],
                                out_hbm.at[pl.ds(cid * h + sid * rps, rps)])
            plsc.subcore_barrier()

        seg_phase(rui_r, rui_c, rui_v, item_emb, o_rui_ei, h_u, p_rui, False)
        seg_phase(rgu_c, rgu_r, rgu_v, group_emb, o_rgu_t_eg, h_u, p_rgu, False)
        seg_phase(rui_c, rui_r, rui_v, user_emb, o_rui_t_eu, h_i, p_rui, True)
        seg_phase(rgi_c, rgi_r, rgi_v, group_emb, o_rgi_t_eg, h_i, p_rgi, True)
        seg_phase(rgi_r, rgi_c, rgi_v, item_emb, o_rgi_ei, h_g, p_rgi, True)
        seg_phase(rgu_r, rgu_c, rgu_v, user_emb, o_rgu_eu, h_g, p_rgu, True)

    return front_kernel


def _pad_edges(rows, cols, vals, mult, n_r, n_c):
    nnz = rows.shape[0]
    nnz_pad = _ceil_to(nnz, mult)
    pad = nnz_pad - nnz
    if pad:
        pad_ids = jnp.arange(pad, dtype=jnp.int32)
        rows = jnp.concatenate([rows, pad_ids % n_r])
        cols = jnp.concatenate([cols, pad_ids % n_c])
        vals = jnp.concatenate([vals, jnp.zeros((pad,), jnp.float32)])
    return rows, cols, vals, nnz_pad


def _segsum(scat_idx, gath_idx, vals, table, n_rows):
    edge_part = n_rows <= 12288
    scat_idx, gath_idx, vals, nnz_pad = _pad_edges(
        scat_idx, gath_idx, vals, 2 * CH * (NC * NS if edge_part else NS),
        n_rows, table.shape[0])
    if edge_part:
        h = _ceil_to(n_rows, RB)
        out = _make_segsum(nnz_pad, h, True)(scat_idx, gath_idx, vals, table)
        return [out[0], out[1]]
    h = _ceil_to(_ceil_to(n_rows, 2 * RB) // 2, 128)
    out = _make_segsum(nnz_pad, h, False)(scat_idx, gath_idx, vals, table)
    return [out]


IB = 1024


def _att_colsum(item, me_pad, n_items):
    mp, _ = me_pad.shape
    nb = _ceil_to(n_items, IB) // IB

    def body(x_ref, me_ref, s_ref):
        @pl.when(pl.program_id(0) == 0)
        def _():
            s_ref[...] = jnp.zeros_like(s_ref)
        e = lax.dot_general(x_ref[...], me_ref[...], (((1,), (1,)), ((), ())),
                            preferred_element_type=jnp.float32)
        rid = pl.program_id(0) * IB + lax.broadcasted_iota(jnp.int32, (IB, 1), 0)
        contrib = jnp.where(rid < n_items, jnp.exp(e), 0.0)
        s_ref[...] += jnp.sum(contrib, axis=0, keepdims=True)

    return pl.pallas_call(
        body,
        grid=(nb,),
        in_specs=[pl.BlockSpec((IB, D), lambda i: (i, 0)),
                  pl.BlockSpec((mp, D), lambda i: (0, 0))],
        out_specs=pl.BlockSpec((1, mp), lambda i: (0, 0)),
        out_shape=jax.ShapeDtypeStruct((1, mp), jnp.float32),
    )(item, me_pad)


def _att_apply(item, me_pad, colsum):
    n_items, _ = item.shape
    mp, _ = me_pad.shape
    nb = _ceil_to(n_items, IB) // IB

    def body(x_ref, me_ref, s_ref, o_ref):
        x = x_ref[...]
        e = lax.dot_general(x, me_ref[...], (((1,), (1,)), ((), ())),
                            preferred_element_type=jnp.float32)
        w = jnp.exp(e) / s_ref[...]
        att = lax.dot_general(w, me_ref[...], (((1,), (0,)), ((), ())),
                              preferred_element_type=jnp.float32)
        o_ref[...] = att * x

    return pl.pallas_call(
        body,
        grid=(nb,),
        in_specs=[pl.BlockSpec((IB, D), lambda i: (i, 0)),
                  pl.BlockSpec((mp, D), lambda i: (0, 0)),
                  pl.BlockSpec((1, mp), lambda i: (0, 0))],
        out_specs=pl.BlockSpec((IB, D), lambda i: (i, 0)),
        out_shape=jax.ShapeDtypeStruct((n_items, D), jnp.float32),
    )(item, me_pad, colsum)


RB = 512


def _finish(acc):
    y = jnp.where(acc >= 0, acc, 0.01 * acc)
    nrm = jnp.sqrt(jnp.sum(y * y, axis=1, keepdims=True))
    return y / jnp.maximum(nrm, 1e-12)


def _dlin(x, w_ref, k):
    return lax.dot_general(x, w_ref[k], (((1,), (1,)), ((), ())),
                           preferred_element_type=jnp.float32)


def _combine(base, part_lists, W, bias, n_rows, group_pattern):
    counts = [len(pl_) for pl_ in part_lists]

    def body(*refs):
        x = refs[0][...]
        pos = 1
        aggs = []
        for c in counts:
            agg = refs[pos][...]
            for r in refs[pos + 1:pos + c]:
                agg = agg + r[...]
            aggs.append(agg)
            pos += c
        w_ref, bias_ref, o_ref = refs[pos], refs[pos + 1], refs[pos + 2]
        acc = jnp.sum(bias_ref[...], axis=0, keepdims=True)
        if group_pattern:
            a, b, c = aggs
            acc = (acc + _dlin(x, w_ref, 0) + _dlin(a, w_ref, 1)
                   + _dlin(b * x, w_ref, 2) + _dlin(a * x, w_ref, 3)
                   + _dlin(c, w_ref, 4))
        else:
            a, b = aggs
            acc = (acc + _dlin(x, w_ref, 0) + _dlin(a, w_ref, 1)
                   + _dlin(a * x, w_ref, 2) + _dlin(b * x, w_ref, 3)
                   + _dlin(b, w_ref, 4))
        o_ref[...] = _finish(acc)

    flat_parts = [p for pl_ in part_lists for p in pl_]
    n_data = 1 + len(flat_parts)
    out = pl.pallas_call(
        body,
        grid=(_ceil_to(n_rows, RB) // RB,),
        in_specs=[pl.BlockSpec((RB, D), lambda i: (i, 0))] * n_data
        + [pl.BlockSpec((5, D, D), lambda i: (0, 0, 0)),
           pl.BlockSpec((5, D), lambda i: (0, 0))],
        out_specs=pl.BlockSpec((RB, D), lambda i: (i, 0)),
        out_shape=jax.ShapeDtypeStruct((n_rows, D), jnp.float32),
    )(base, *flat_parts, W, bias)
    return out


def kernel(group_embedding, user_embedding, item_embedding, members,
           rui_rows, rui_cols, rui_vals, rgu_rows, rgu_cols, rgu_vals,
           rgi_rows, rgi_cols, rgi_vals, Wg, bg, Wu, bu, Wi, bi):
    G, U, I = group_embedding.shape[0], user_embedding.shape[0], item_embedding.shape[0]

    h_u = _ceil_to(_ceil_to(U, 2 * RB) // 2, 128)
    h_i = _ceil_to(I, RB)
    h_g = _ceil_to(G, RB)
    rui_r, rui_c, rui_v, p_rui = _pad_edges(rui_rows, rui_cols, rui_vals, 2 * CH * NC * NS, U, I)
    rgu_r, rgu_c, rgu_v, p_rgu = _pad_edges(rgu_rows, rgu_cols, rgu_vals, 2 * CH * NC * NS, G, U)
    rgi_r, rgi_c, rgi_v, p_rgi = _pad_edges(rgi_rows, rgi_cols, rgi_vals, 2 * CH * NC * NS, G, I)
    mflat = members.reshape(-1).astype(jnp.int32)
    n_mem = mflat.shape[0]
    mp = _ceil_to(n_mem, 8 * NC * NS)
    if mp != n_mem:
        mflat = jnp.concatenate([mflat, jnp.zeros((mp - n_mem,), jnp.int32)])

    front = _make_front(p_rui, p_rgu, p_rgi, h_u, h_i, h_g, mp)
    (rui_ei_f, rgu_t_eg_f, rui_t_eu_p, rgi_t_eg_p, rgi_ei_p, rgu_eu_p, me) = front(
        rui_r, rui_c, rui_v, rgu_r, rgu_c, rgu_v, rgi_r, rgi_c, rgi_v,
        item_embedding, user_embedding, group_embedding, mflat)
    rui_ei = [rui_ei_f]
    rgu_t_eg = [rgu_t_eg_f]
    rui_t_eu = [rui_t_eu_p[0], rui_t_eu_p[1]]
    rgi_t_eg = [rgi_t_eg_p[0], rgi_t_eg_p[1]]
    rgi_ei = [rgi_ei_p[0], rgi_ei_p[1]]
    rgu_eu = [rgu_eu_p[0], rgu_eu_p[1]]
    me_pad = jnp.where(
        (jnp.arange(mp) < n_mem)[:, None], me, 0.0)
    colsum = _att_colsum(item_embedding, me_pad, I)
    attentive = _att_apply(item_embedding, me_pad, colsum)

    atten_g = _segsum(rgi_rows, rgi_cols, rgi_vals, attentive, G)

    nu = _combine(user_embedding, [rui_ei, rgu_t_eg], Wu, bu, U, False)
    ni = _combine(item_embedding, [rui_t_eu, rgi_t_eg], Wi, bi, I, False)
    ng = _combine(group_embedding, [rgi_ei, rgu_eu, atten_g], Wg, bg, G, True)
    return ng, nu, ni
